# Initial kernel scaffold; baseline (speedup 1.0000x reference)
import functools

import jax
import jax.numpy as jnp
from jax import lax
from jax.experimental import pallas as pl
from jax.experimental.pallas import tpu as pltpu
from jax.experimental.pallas import tpu_sc as plsc

N = 10000
E = 320000
D = 128
R = 16
NB = 8

NC = 2
NS = 16
NW = NC * NS
EPT = E // NW
C = 80
NCH = EPT // C
RPT = N // NS
ZB = 125

BN = 400
NBLK = N // BN

_MESH = plsc.VectorSubcoreMesh(core_axis_name="c", subcore_axis_name="s")
_F32 = jnp.float32


def _worker_ids():
    c = lax.axis_index("c")
    s = lax.axis_index("s")
    return c, s, s * NC + c


def _sc_count_body(dst_h, typ_h, out_h, dstv, typv, oh, zb, acc):
    c, s, wid = _worker_ids()
    zeros16 = jnp.zeros((16,), _F32)

    @pl.loop(0, ZB)
    def _(i):
        zb[i] = zeros16

    @pl.loop(0, RPT // ZB)
    def _(i):
        pltpu.sync_copy(zb, acc.at[pl.ds(s * RPT + i * ZB, ZB)])

    plsc.subcore_barrier()

    base0 = wid * EPT
    iot = lax.iota(jnp.int32, 16)

    @pl.loop(0, NCH)
    def _(i):
        base = base0 + i * C
        pltpu.sync_copy(dst_h.at[pl.ds(base, C)], dstv)
        pltpu.sync_copy(typ_h.at[pl.ds(base, C)], typv)

        @pl.loop(0, C)
        def _(e):
            t = typv[e]
            oh[e] = jnp.where(iot == t, 1.0, 0.0).astype(_F32)

        pltpu.sync_copy(oh, acc.at[dstv], add=True)

    plsc.subcore_barrier()
    pltpu.sync_copy(acc.at[pl.ds(s * RPT, RPT)],
                    out_h.at[c, pl.ds(s * RPT, RPT)])


def _sc_count(dst, typ):
    return pl.kernel(
        _sc_count_body,
        out_type=jax.ShapeDtypeStruct((NC, N, R), _F32),
        mesh=_MESH,
        scratch_types=[
            pltpu.VMEM((C,), jnp.int32),
            pltpu.VMEM((C,), jnp.int32),
            pltpu.VMEM((C, R), _F32),
            pltpu.VMEM((ZB, R), _F32),
            pltpu.VMEM_SHARED((N, R), _F32),
        ],
    )(dst, typ)


def _sc_wg_body(winv_h, src_h, dst_h, typ_h, w_h, g_h,
                srcv, dstv, typv, rows, wv, gv, sem):
    c, s, wid = _worker_ids()
    base0 = wid * EPT
    iot = lax.iota(jnp.int32, 16)

    @pl.loop(0, NCH)
    def _(i):
        base = base0 + i * C
        pltpu.sync_copy(src_h.at[pl.ds(base, C)], srcv)
        pltpu.sync_copy(dst_h.at[pl.ds(base, C)], dstv)
        pltpu.sync_copy(typ_h.at[pl.ds(base, C)], typv)
        pltpu.async_copy(winv_h.at[dstv], rows, sem).wait()
        for k in range(C // 16):
            sl = pl.ds(16 * k, 16)
            tt = typv[sl]
            ww = plsc.load_gather(rows, [iot + 16 * k, tt])
            wv[sl] = ww
            gv[sl] = tt * N + srcv[sl]
        pltpu.sync_copy(wv, w_h.at[pl.ds(base, C)])
        pltpu.sync_copy(gv, g_h.at[pl.ds(base, C)])


def _sc_wg(winv, src, dst, typ):
    return pl.kernel(
        _sc_wg_body,
        out_type=(jax.ShapeDtypeStruct((E,), _F32),
                  jax.ShapeDtypeStruct((E,), jnp.int32)),
        mesh=_MESH,
        scratch_types=[
            pltpu.VMEM((C,), jnp.int32),
            pltpu.VMEM((C,), jnp.int32),
            pltpu.VMEM((C,), jnp.int32),
            pltpu.VMEM((C, R), _F32),
            pltpu.VMEM((C,), _F32),
            pltpu.VMEM((C,), jnp.int32),
            pltpu.SemaphoreType.DMA,
        ],
    )(winv, src, dst, typ)


def _sc_layer_body(y_h, g_h, w_h, dst_h, out_h,
                   gv, wv, dstv, ybuf, zb, acc, sem):
    c, s, wid = _worker_ids()
    zeros16 = jnp.zeros((16,), _F32)

    @pl.loop(0, ZB)
    def _(i):
        for j in range(D // 16):
            zb[i, pl.ds(16 * j, 16)] = zeros16

    @pl.loop(0, RPT // ZB)
    def _(i):
        pltpu.sync_copy(zb, acc.at[pl.ds(s * RPT + i * ZB, ZB)])

    plsc.subcore_barrier()

    base0 = wid * EPT

    @pl.loop(0, NCH)
    def _(i):
        base = base0 + i * C
        pltpu.sync_copy(g_h.at[pl.ds(base, C)], gv)
        pltpu.sync_copy(w_h.at[pl.ds(base, C)], wv)
        pltpu.sync_copy(dst_h.at[pl.ds(base, C)], dstv)
        pltpu.async_copy(y_h.at[gv], ybuf, sem).wait()

        @pl.loop(0, C)
        def _(e):
            we = wv[e]
            for j in range(D // 16):
                sl = pl.ds(16 * j, 16)
                ybuf[e, sl] = ybuf[e, sl] * we

        pltpu.sync_copy(ybuf, acc.at[dstv], add=True)

    plsc.subcore_barrier()
    pltpu.sync_copy(acc.at[pl.ds(s * RPT, RPT)],
                    out_h.at[c, pl.ds(s * RPT, RPT)])


def _sc_layer(yflat, g, w, dst):
    return pl.kernel(
        _sc_layer_body,
        out_type=jax.ShapeDtypeStruct((NC, N, D), _F32),
        mesh=_MESH,
        scratch_types=[
            pltpu.VMEM((C,), jnp.int32),
            pltpu.VMEM((C,), _F32),
            pltpu.VMEM((C,), jnp.int32),
            pltpu.VMEM((C, D), _F32),
            pltpu.VMEM((ZB, D), _F32),
            pltpu.VMEM_SHARED((N, D), _F32),
            pltpu.SemaphoreType.DMA,
        ],
    )(yflat, g, w, dst)


def _tc_weights_body(comp_ref, basis_ref, out_ref):
    out_ref[0] = jnp.dot(comp_ref[0], basis_ref[0],
                         preferred_element_type=_F32)


def _tc_weights(comp_all, basis_all):
    return pl.pallas_call(
        _tc_weights_body,
        grid=(3,),
        in_specs=[
            pl.BlockSpec((1, R, NB), lambda i: (i, 0, 0)),
            pl.BlockSpec((1, NB, D * D), lambda i: (i, 0, 0)),
        ],
        out_specs=pl.BlockSpec((1, R, D * D), lambda i: (i, 0, 0)),
        out_shape=jax.ShapeDtypeStruct((3, R, D * D), _F32),
    )(comp_all, basis_all)


def _tc_mm_body(h_ref, w_ref, y_ref):
    h = h_ref[...]
    for r in range(R):
        y_ref[r] = jnp.dot(h, w_ref[r], preferred_element_type=_F32)


def _tc_mm(h, w3):
    return pl.pallas_call(
        _tc_mm_body,
        grid=(NBLK,),
        in_specs=[
            pl.BlockSpec((BN, D), lambda j: (j, 0)),
            pl.BlockSpec((R, D, D), lambda j: (0, 0, 0)),
        ],
        out_specs=pl.BlockSpec((R, BN, D), lambda j: (0, j, 0)),
        out_shape=jax.ShapeDtypeStruct((R, N, D), _F32),
    )(h, w3)


def _tc_winv_body(cnt_ref, out_ref):
    csum = cnt_ref[0] + cnt_ref[1]
    out_ref[...] = 1.0 / jnp.maximum(csum, 1.0)


def _tc_winv(cnts):
    return pl.pallas_call(
        _tc_winv_body,
        out_shape=jax.ShapeDtypeStruct((N, R), _F32),
    )(cnts)


def _tc_dense_body(h_ref, root_ref, bias_ref, acc_ref, out_ref, *, relu):
    o = jnp.dot(h_ref[...], root_ref[...], preferred_element_type=_F32)
    o = o + bias_ref[...] + acc_ref[0] + acc_ref[1]
    if relu:
        o = jnp.maximum(o, 0.0)
    out_ref[...] = o


def _tc_dense(h, root, bias, acc, relu):
    return pl.pallas_call(
        functools.partial(_tc_dense_body, relu=relu),
        grid=(NBLK,),
        in_specs=[
            pl.BlockSpec((BN, D), lambda j: (j, 0)),
            pl.BlockSpec((D, D), lambda j: (0, 0)),
            pl.BlockSpec((1, D), lambda j: (0, 0)),
            pl.BlockSpec((NC, BN, D), lambda j: (0, j, 0)),
        ],
        out_specs=pl.BlockSpec((BN, D), lambda j: (j, 0)),
        out_shape=jax.ShapeDtypeStruct((N, D), _F32),
    )(h, root, bias, acc)


def kernel(x, edge_index, edge_type,
           comp1, basis1, root1, bias1,
           comp2, basis2, root2, bias2,
           comp3, basis3, root3, bias3):
    src = edge_index[0]
    dst = edge_index[1]

    comp_all = jnp.stack([comp1, comp2, comp3])
    basis_all = jnp.stack([basis1.reshape(NB, D * D),
                           basis2.reshape(NB, D * D),
                           basis3.reshape(NB, D * D)])
    w_all = _tc_weights(comp_all, basis_all).reshape(3, R, D, D)

    cnts = _sc_count(dst, edge_type)
    winv = _tc_winv(cnts)
    w_e, g_e = _sc_wg(winv, src, dst, edge_type)

    roots = (root1, root2, root3)
    biases = (bias1, bias2, bias3)
    h = x
    for i in range(3):
        y = _tc_mm(h, w_all[i])
        acc = _sc_layer(y.reshape(R * N, D), g_e, w_e, dst)
        h = _tc_dense(h, roots[i], biases[i].reshape(1, D), acc,
                      relu=(i < 2))
    return h


# speedup vs baseline: 21.7338x; 21.7338x over previous
import functools

import jax
import jax.numpy as jnp
from jax import lax
from jax.experimental import pallas as pl
from jax.experimental.pallas import tpu as pltpu
from jax.experimental.pallas import tpu_sc as plsc

N = 10000
E = 320000
D = 128
R = 16
NB = 8

NC = 2
NS = 16
NW = NC * NS
EPT = E // NW
C = 80
NCH = EPT // C
NPAD = 10240
RPT = NPAD // NS
ZB = 128

BN = 400
NBLK = N // BN

_MESH = plsc.VectorSubcoreMesh(core_axis_name="c", subcore_axis_name="s")
_SC_PARAMS = pltpu.CompilerParams(needs_layout_passes=False)
_F32 = jnp.float32


def _worker_ids():
    c = lax.axis_index("c")
    s = lax.axis_index("s")
    return c, s, s * NC + c


def _sc_count_body(dst_h, typ_h, out_h, dstv, typv, oh, zb, acc):
    c, s, wid = _worker_ids()
    zeros16 = jnp.zeros((16,), _F32)

    @pl.loop(0, ZB)
    def _(i):
        for j in range(D // 16):
            zb[i, pl.ds(16 * j, 16)] = zeros16

    @pl.loop(0, C)
    def _(e):
        for j in range(D // 16):
            oh[e, pl.ds(16 * j, 16)] = zeros16

    @pl.loop(0, RPT // ZB)
    def _(i):
        pltpu.sync_copy(zb, acc.at[pl.ds(s * RPT + i * ZB, ZB)])

    plsc.subcore_barrier()

    base0 = wid * EPT
    iot = lax.iota(jnp.int32, 16)

    @pl.loop(0, NCH)
    def _(i):
        base = base0 + i * C
        pltpu.sync_copy(dst_h.at[pl.ds(base, C)], dstv)
        pltpu.sync_copy(typ_h.at[pl.ds(base, C)], typv)

        @pl.loop(0, C)
        def _(e):
            t = plsc.load_gather(typv, [jnp.full((16,), e, jnp.int32)])
            oh[e, pl.ds(0, 16)] = jnp.where(iot == t, 1.0, 0.0).astype(_F32)

        pltpu.sync_copy(oh, acc.at[dstv], add=True)

    plsc.subcore_barrier()
    pltpu.sync_copy(acc.at[pl.ds(s * RPT, RPT)],
                    out_h.at[c, pl.ds(s * RPT, RPT)])


def _sc_count(dst, typ):
    return pl.kernel(
        _sc_count_body,
        out_type=jax.ShapeDtypeStruct((NC, NPAD, D), _F32),
        mesh=_MESH,
        compiler_params=_SC_PARAMS,
        scratch_types=[
            pltpu.VMEM((C,), jnp.int32),
            pltpu.VMEM((C,), jnp.int32),
            pltpu.VMEM((C, D), _F32),
            pltpu.VMEM((ZB, D), _F32),
            pltpu.VMEM_SHARED((NPAD, D), _F32),
        ],
    )(dst, typ)


def _sc_wg_body(winv_h, src_h, dst_h, typ_h, w_h, g_h,
                srcv, dstv, typv, rows, wv, gv, sem):
    c, s, wid = _worker_ids()
    base0 = wid * EPT
    iot = lax.iota(jnp.int32, 16)

    @pl.loop(0, NCH)
    def _(i):
        base = base0 + i * C
        pltpu.sync_copy(src_h.at[pl.ds(base, C)], srcv)
        pltpu.sync_copy(dst_h.at[pl.ds(base, C)], dstv)
        pltpu.sync_copy(typ_h.at[pl.ds(base, C)], typv)
        pltpu.async_copy(winv_h.at[dstv], rows, sem).wait()
        for k in range(C // 16):
            sl = pl.ds(16 * k, 16)
            tt = typv[sl]
            ww = plsc.load_gather(rows, [iot + 16 * k, tt])
            wv[sl] = ww
            gv[sl] = tt * N + srcv[sl]
        pltpu.sync_copy(wv, w_h.at[pl.ds(base, C)])
        pltpu.sync_copy(gv, g_h.at[pl.ds(base, C)])


def _sc_wg(winv, src, dst, typ):
    return pl.kernel(
        _sc_wg_body,
        out_type=(jax.ShapeDtypeStruct((E,), _F32),
                  jax.ShapeDtypeStruct((E,), jnp.int32)),
        mesh=_MESH,
        compiler_params=_SC_PARAMS,
        scratch_types=[
            pltpu.VMEM((C,), jnp.int32),
            pltpu.VMEM((C,), jnp.int32),
            pltpu.VMEM((C,), jnp.int32),
            pltpu.VMEM((C, 128), _F32),
            pltpu.VMEM((C,), _F32),
            pltpu.VMEM((C,), jnp.int32),
            pltpu.SemaphoreType.DMA,
        ],
    )(winv, src, dst, typ)


def _sc_layer_body(y_h, g_h, w_h, dst_h, out_h,
                   gv, wv, dstv, ybuf, zb, acc, sem):
    c, s, wid = _worker_ids()
    zeros16 = jnp.zeros((16,), _F32)

    @pl.loop(0, ZB)
    def _(i):
        for j in range(D // 16):
            zb[i, pl.ds(16 * j, 16)] = zeros16

    @pl.loop(0, RPT // ZB)
    def _(i):
        pltpu.sync_copy(zb, acc.at[pl.ds(s * RPT + i * ZB, ZB)])

    plsc.subcore_barrier()

    base0 = wid * EPT

    @pl.loop(0, NCH)
    def _(i):
        base = base0 + i * C
        pltpu.sync_copy(g_h.at[pl.ds(base, C)], gv)
        pltpu.sync_copy(w_h.at[pl.ds(base, C)], wv)
        pltpu.sync_copy(dst_h.at[pl.ds(base, C)], dstv)
        pltpu.async_copy(y_h.at[gv], ybuf, sem).wait()

        @pl.loop(0, C)
        def _(e):
            we = plsc.load_gather(wv, [jnp.full((16,), e, jnp.int32)])
            for j in range(D // 16):
                sl = pl.ds(16 * j, 16)
                ybuf[e, sl] = ybuf[e, sl] * we

        pltpu.sync_copy(ybuf, acc.at[dstv], add=True)

    plsc.subcore_barrier()
    pltpu.sync_copy(acc.at[pl.ds(s * RPT, RPT)],
                    out_h.at[c, pl.ds(s * RPT, RPT)])


def _sc_layer(yflat, g, w, dst):
    return pl.kernel(
        _sc_layer_body,
        out_type=jax.ShapeDtypeStruct((NC, NPAD, D), _F32),
        mesh=_MESH,
        compiler_params=_SC_PARAMS,
        scratch_types=[
            pltpu.VMEM((C,), jnp.int32),
            pltpu.VMEM((C,), _F32),
            pltpu.VMEM((C,), jnp.int32),
            pltpu.VMEM((C, D), _F32),
            pltpu.VMEM((ZB, D), _F32),
            pltpu.VMEM_SHARED((NPAD, D), _F32),
            pltpu.SemaphoreType.DMA,
        ],
    )(yflat, g, w, dst)


def _tc_weights_body(comp_ref, basis_ref, out_ref):
    out_ref[0] = jnp.dot(comp_ref[0], basis_ref[0],
                         preferred_element_type=_F32)


def _tc_weights(comp_all, basis_all):
    return pl.pallas_call(
        _tc_weights_body,
        grid=(3,),
        in_specs=[
            pl.BlockSpec((1, R, NB), lambda i: (i, 0, 0)),
            pl.BlockSpec((1, NB, D * D), lambda i: (i, 0, 0)),
        ],
        out_specs=pl.BlockSpec((1, R, D * D), lambda i: (i, 0, 0)),
        out_shape=jax.ShapeDtypeStruct((3, R, D * D), _F32),
    )(comp_all, basis_all)


def _tc_mm_body(h_ref, w_ref, y_ref):
    h = h_ref[...]
    for r in range(R):
        y_ref[r] = jnp.dot(h, w_ref[r], preferred_element_type=_F32)


def _tc_mm(h, w3):
    return pl.pallas_call(
        _tc_mm_body,
        grid=(NBLK,),
        in_specs=[
            pl.BlockSpec((BN, D), lambda j: (j, 0)),
            pl.BlockSpec((R, D, D), lambda j: (0, 0, 0)),
        ],
        out_specs=pl.BlockSpec((R, BN, D), lambda j: (0, j, 0)),
        out_shape=jax.ShapeDtypeStruct((R, N, D), _F32),
    )(h, w3)


def _tc_winv_body(cnt_ref, out_ref):
    csum = cnt_ref[0] + cnt_ref[1]
    out_ref[...] = 1.0 / jnp.maximum(csum, 1.0)


def _tc_winv(cnts):
    return pl.pallas_call(
        _tc_winv_body,
        grid=(NS,),
        in_specs=[pl.BlockSpec((NC, RPT, D), lambda j: (0, j, 0))],
        out_specs=pl.BlockSpec((RPT, D), lambda j: (j, 0)),
        out_shape=jax.ShapeDtypeStruct((NPAD, D), _F32),
    )(cnts)


def _tc_dense_body(h_ref, root_ref, bias_ref, acc_ref, out_ref, *, relu):
    o = jnp.dot(h_ref[...], root_ref[...], preferred_element_type=_F32)
    o = o + bias_ref[...] + acc_ref[0] + acc_ref[1]
    if relu:
        o = jnp.maximum(o, 0.0)
    out_ref[...] = o


def _tc_dense(h, root, bias, acc, relu):
    return pl.pallas_call(
        functools.partial(_tc_dense_body, relu=relu),
        grid=(NBLK,),
        in_specs=[
            pl.BlockSpec((BN, D), lambda j: (j, 0)),
            pl.BlockSpec((D, D), lambda j: (0, 0)),
            pl.BlockSpec((1, D), lambda j: (0, 0)),
            pl.BlockSpec((NC, BN, D), lambda j: (0, j, 0)),
        ],
        out_specs=pl.BlockSpec((BN, D), lambda j: (j, 0)),
        out_shape=jax.ShapeDtypeStruct((N, D), _F32),
    )(h, root, bias, acc)


def kernel(x, edge_index, edge_type,
           comp1, basis1, root1, bias1,
           comp2, basis2, root2, bias2,
           comp3, basis3, root3, bias3):
    src = edge_index[0]
    dst = edge_index[1]

    comp_all = jnp.stack([comp1, comp2, comp3])
    basis_all = jnp.stack([basis1.reshape(NB, D * D),
                           basis2.reshape(NB, D * D),
                           basis3.reshape(NB, D * D)])
    w_all = _tc_weights(comp_all, basis_all).reshape(3, R, D, D)

    cnts = _sc_count(dst, edge_type)
    winv = _tc_winv(cnts)
    w_e, g_e = _sc_wg(winv, src, dst, edge_type)

    roots = (root1, root2, root3)
    biases = (bias1, bias2, bias3)
    h = x
    for i in range(3):
        y = _tc_mm(h, w_all[i])
        acc = _sc_layer(y.reshape(R * N, D), g_e, w_e, dst)
        h = _tc_dense(h, roots[i], biases[i].reshape(1, D), acc,
                      relu=(i < 2))
    return h


# speedup vs baseline: 27.0283x; 1.2436x over previous
import functools

import jax
import jax.numpy as jnp
from jax import lax
from jax.experimental import pallas as pl
from jax.experimental.pallas import tpu as pltpu
from jax.experimental.pallas import tpu_sc as plsc

N = 10000
E = 320000
D = 128
R = 16
NB = 8

NC = 2
NS = 16
NW = NC * NS
EPT = E // NW
C = 80
NCH = EPT // C
NBUF = 4
RING = NCH - 1
NPAD = 10240
RPT = NPAD // NS
ZB = 32

BN = 400
NBLK = N // BN

_MESH = plsc.VectorSubcoreMesh(core_axis_name="c", subcore_axis_name="s")
_SC_PARAMS = pltpu.CompilerParams(needs_layout_passes=False)
_F32 = jnp.float32


def _worker_ids():
    c = lax.axis_index("c")
    s = lax.axis_index("s")
    return c, s, s * NC + c


def _sc_count_body(dst_h, typ_h, out_h, dstv, typv, oh, zb, acc):
    c, s, wid = _worker_ids()
    zeros16 = jnp.zeros((16,), _F32)

    @pl.loop(0, ZB)
    def _(i):
        for j in range(D // 16):
            zb[i, pl.ds(16 * j, 16)] = zeros16

    @pl.loop(0, C)
    def _(e):
        for j in range(D // 16):
            oh[e, pl.ds(16 * j, 16)] = zeros16

    @pl.loop(0, RPT // ZB)
    def _(i):
        pltpu.sync_copy(zb, acc.at[pl.ds(s * RPT + i * ZB, ZB)])

    plsc.subcore_barrier()

    base0 = wid * EPT
    iot = lax.iota(jnp.int32, 16)

    @pl.loop(0, NCH)
    def _(i):
        base = base0 + i * C
        pltpu.sync_copy(dst_h.at[pl.ds(base, C)], dstv)
        pltpu.sync_copy(typ_h.at[pl.ds(base, C)], typv)

        @pl.loop(0, C)
        def _(e):
            t = plsc.load_gather(typv, [jnp.full((16,), e, jnp.int32)])
            oh[e, pl.ds(0, 16)] = jnp.where(iot == t, 1.0, 0.0).astype(_F32)

        pltpu.sync_copy(oh, acc.at[dstv], add=True)

    plsc.subcore_barrier()
    pltpu.sync_copy(acc.at[pl.ds(s * RPT, RPT)],
                    out_h.at[c, pl.ds(s * RPT, RPT)])


def _sc_count(dst, typ):
    return pl.kernel(
        _sc_count_body,
        out_type=jax.ShapeDtypeStruct((NC, NPAD, D), _F32),
        mesh=_MESH,
        compiler_params=_SC_PARAMS,
        scratch_types=[
            pltpu.VMEM((C,), jnp.int32),
            pltpu.VMEM((C,), jnp.int32),
            pltpu.VMEM((C, D), _F32),
            pltpu.VMEM((ZB, D), _F32),
            pltpu.VMEM_SHARED((NPAD, D), _F32),
        ],
    )(dst, typ)


def _sc_wg_body(winv_h, src_h, dst_h, typ_h, w_h, g_h,
                srcv, dstv, typv, rows, wv, gv, sem):
    c, s, wid = _worker_ids()
    base0 = wid * EPT
    iot = lax.iota(jnp.int32, 16)

    @pl.loop(0, NCH)
    def _(i):
        base = base0 + i * C
        pltpu.sync_copy(src_h.at[pl.ds(base, C)], srcv)
        pltpu.sync_copy(dst_h.at[pl.ds(base, C)], dstv)
        pltpu.sync_copy(typ_h.at[pl.ds(base, C)], typv)
        pltpu.async_copy(winv_h.at[dstv], rows, sem).wait()
        for k in range(C // 16):
            sl = pl.ds(16 * k, 16)
            tt = typv[sl]
            ww = plsc.load_gather(rows, [iot + 16 * k, tt])
            wv[sl] = ww
            gv[sl] = tt * N + srcv[sl]
        pltpu.sync_copy(wv, w_h.at[pl.ds(base, C)])
        pltpu.sync_copy(gv, g_h.at[pl.ds(base, C)])


def _sc_wg(winv, src, dst, typ):
    return pl.kernel(
        _sc_wg_body,
        out_type=(jax.ShapeDtypeStruct((E,), _F32),
                  jax.ShapeDtypeStruct((E,), jnp.int32)),
        mesh=_MESH,
        compiler_params=_SC_PARAMS,
        scratch_types=[
            pltpu.VMEM((C,), jnp.int32),
            pltpu.VMEM((C,), jnp.int32),
            pltpu.VMEM((C,), jnp.int32),
            pltpu.VMEM((C, 128), _F32),
            pltpu.VMEM((C,), _F32),
            pltpu.VMEM((C,), jnp.int32),
            pltpu.SemaphoreType.DMA,
        ],
    )(winv, src, dst, typ)


def _sc_layer_body(y_h, g_h, w_h, dst_h, out_h, zb, acc, *rest):
    bufs = [rest[4 * b:4 * b + 4] for b in range(NBUF)]
    gsems = rest[4 * NBUF:5 * NBUF]
    ssems = rest[5 * NBUF:6 * NBUF]
    c, s, wid = _worker_ids()
    zeros16 = jnp.zeros((16,), _F32)

    @pl.loop(0, ZB)
    def _(i):
        for j in range(D // 16):
            zb[i, pl.ds(16 * j, 16)] = zeros16

    @pl.loop(0, RPT // ZB)
    def _(i):
        pltpu.sync_copy(zb, acc.at[pl.ds(s * RPT + i * ZB, ZB)])

    plsc.subcore_barrier()

    base0 = wid * EPT

    def load_meta(b, i):
        gv, wv, dstv, yb = bufs[b]
        base = base0 + i * C
        pltpu.sync_copy(g_h.at[pl.ds(base, C)], gv)
        pltpu.sync_copy(w_h.at[pl.ds(base, C)], wv)
        pltpu.sync_copy(dst_h.at[pl.ds(base, C)], dstv)

    for b in range(NBUF):
        load_meta(b, b)
        gv, wv, dstv, yb = bufs[b]
        pltpu.async_copy(y_h.at[gv], yb, gsems[b])

    def scale(wv, yb):
        @pl.loop(0, C, unroll=8)
        def _(e):
            we = plsc.load_gather(wv, [jnp.full((16,), e, jnp.int32)])
            for j in range(D // 16):
                sl = pl.ds(16 * j, 16)
                yb[e, sl] = yb[e, sl] * we

    @pl.loop(0, RING, step=NBUF)
    def _(p):
        for b in range(NBUF):
            i = p + b
            gv, wv, dstv, yb = bufs[b]
            pltpu.make_async_copy(y_h.at[gv], yb, gsems[b]).wait()
            scale(wv, yb)
            pltpu.async_copy(yb, acc.at[dstv], ssems[b], add=True)

            @pl.when(i + NBUF < RING)
            def _():
                pltpu.make_async_copy(yb, acc.at[dstv], ssems[b]).wait()
                load_meta(b, i + NBUF)
                pltpu.async_copy(y_h.at[gv], yb, gsems[b])

    gv, wv, dstv, yb = bufs[0]
    pltpu.make_async_copy(yb, acc.at[dstv], ssems[0]).wait()
    load_meta(0, RING)
    pltpu.async_copy(y_h.at[gv], yb, gsems[0]).wait()
    scale(wv, yb)
    pltpu.sync_copy(yb, acc.at[dstv], add=True)

    for b in range(1, NBUF):
        gv, wv, dstv, yb = bufs[b]
        pltpu.make_async_copy(yb, acc.at[dstv], ssems[b]).wait()

    plsc.subcore_barrier()
    pltpu.sync_copy(acc.at[pl.ds(s * RPT, RPT)],
                    out_h.at[c, pl.ds(s * RPT, RPT)])


def _sc_layer(yflat, g, w, dst):
    per_buf = [pltpu.VMEM((C,), jnp.int32),
               pltpu.VMEM((C,), _F32),
               pltpu.VMEM((C,), jnp.int32),
               pltpu.VMEM((C, D), _F32)]
    return pl.kernel(
        _sc_layer_body,
        out_type=jax.ShapeDtypeStruct((NC, NPAD, D), _F32),
        mesh=_MESH,
        compiler_params=_SC_PARAMS,
        scratch_types=(
            [pltpu.VMEM((ZB, D), _F32),
             pltpu.VMEM_SHARED((NPAD, D), _F32)]
            + per_buf * NBUF
            + [pltpu.SemaphoreType.DMA] * (2 * NBUF)
        ),
    )(yflat, g, w, dst)


def _tc_weights_body(comp_ref, basis_ref, out_ref):
    out_ref[0] = jnp.dot(comp_ref[0], basis_ref[0],
                         preferred_element_type=_F32)


def _tc_weights(comp_all, basis_all):
    return pl.pallas_call(
        _tc_weights_body,
        grid=(3,),
        in_specs=[
            pl.BlockSpec((1, R, NB), lambda i: (i, 0, 0)),
            pl.BlockSpec((1, NB, D * D), lambda i: (i, 0, 0)),
        ],
        out_specs=pl.BlockSpec((1, R, D * D), lambda i: (i, 0, 0)),
        out_shape=jax.ShapeDtypeStruct((3, R, D * D), _F32),
    )(comp_all, basis_all)


def _tc_mm_body(h_ref, w_ref, y_ref):
    h = h_ref[...]
    for r in range(R):
        y_ref[r] = jnp.dot(h, w_ref[r], preferred_element_type=_F32)


def _tc_mm(h, w3):
    return pl.pallas_call(
        _tc_mm_body,
        grid=(NBLK,),
        in_specs=[
            pl.BlockSpec((BN, D), lambda j: (j, 0)),
            pl.BlockSpec((R, D, D), lambda j: (0, 0, 0)),
        ],
        out_specs=pl.BlockSpec((R, BN, D), lambda j: (0, j, 0)),
        out_shape=jax.ShapeDtypeStruct((R, N, D), _F32),
    )(h, w3)


def _tc_winv_body(cnt_ref, out_ref):
    csum = cnt_ref[0] + cnt_ref[1]
    out_ref[...] = 1.0 / jnp.maximum(csum, 1.0)


def _tc_winv(cnts):
    return pl.pallas_call(
        _tc_winv_body,
        grid=(NS,),
        in_specs=[pl.BlockSpec((NC, RPT, D), lambda j: (0, j, 0))],
        out_specs=pl.BlockSpec((RPT, D), lambda j: (j, 0)),
        out_shape=jax.ShapeDtypeStruct((NPAD, D), _F32),
    )(cnts)


def _tc_dense_body(h_ref, root_ref, bias_ref, acc_ref, out_ref, *, relu):
    o = jnp.dot(h_ref[...], root_ref[...], preferred_element_type=_F32)
    o = o + bias_ref[...] + acc_ref[0] + acc_ref[1]
    if relu:
        o = jnp.maximum(o, 0.0)
    out_ref[...] = o


def _tc_dense(h, root, bias, acc, relu):
    return pl.pallas_call(
        functools.partial(_tc_dense_body, relu=relu),
        grid=(NBLK,),
        in_specs=[
            pl.BlockSpec((BN, D), lambda j: (j, 0)),
            pl.BlockSpec((D, D), lambda j: (0, 0)),
            pl.BlockSpec((1, D), lambda j: (0, 0)),
            pl.BlockSpec((NC, BN, D), lambda j: (0, j, 0)),
        ],
        out_specs=pl.BlockSpec((BN, D), lambda j: (j, 0)),
        out_shape=jax.ShapeDtypeStruct((N, D), _F32),
    )(h, root, bias, acc)


def kernel(x, edge_index, edge_type,
           comp1, basis1, root1, bias1,
           comp2, basis2, root2, bias2,
           comp3, basis3, root3, bias3):
    src = edge_index[0]
    dst = edge_index[1]

    comp_all = jnp.stack([comp1, comp2, comp3])
    basis_all = jnp.stack([basis1.reshape(NB, D * D),
                           basis2.reshape(NB, D * D),
                           basis3.reshape(NB, D * D)])
    w_all = _tc_weights(comp_all, basis_all).reshape(3, R, D, D)

    cnts = _sc_count(dst, edge_type)
    winv = _tc_winv(cnts)
    w_e, g_e = _sc_wg(winv, src, dst, edge_type)

    roots = (root1, root2, root3)
    biases = (bias1, bias2, bias3)
    h = x
    for i in range(3):
        y = _tc_mm(h, w_all[i])
        acc = _sc_layer(y.reshape(R * N, D), g_e, w_e, dst)
        h = _tc_dense(h, roots[i], biases[i].reshape(1, D), acc,
                      relu=(i < 2))
    return h


# speedup vs baseline: 30.4503x; 1.1266x over previous
import functools

import jax
import jax.numpy as jnp
from jax import lax
from jax.experimental import pallas as pl
from jax.experimental.pallas import tpu as pltpu
from jax.experimental.pallas import tpu_sc as plsc

N = 10000
E = 320000
D = 128
R = 16
NB = 8

NC = 2
NS = 16
NW = NC * NS
EPT = E // NW
C = 80
NCH = EPT // C
NBUF = 4
RING = NCH - 1
NPAD = 10240
RPT = NPAD // NS
ZB = 32

BN = 400
NBLK = N // BN

_MESH = plsc.VectorSubcoreMesh(core_axis_name="c", subcore_axis_name="s")
_SC_PARAMS = pltpu.CompilerParams(needs_layout_passes=False)
_F32 = jnp.float32


def _worker_ids():
    c = lax.axis_index("c")
    s = lax.axis_index("s")
    return c, s, s * NC + c


def _sc_count_body(dst_h, typ_h, out_h, zb, acc, *rest):
    bufs = [rest[3 * b:3 * b + 3] for b in range(NBUF)]
    ssems = rest[3 * NBUF:4 * NBUF]
    c, s, wid = _worker_ids()
    zeros16 = jnp.zeros((16,), _F32)

    @pl.loop(0, ZB)
    def _(i):
        for j in range(D // 16):
            zb[i, pl.ds(16 * j, 16)] = zeros16

    for b in range(NBUF):
        _, _, oh = bufs[b]

        @pl.loop(0, C)
        def _(e):
            for j in range(D // 16):
                oh[e, pl.ds(16 * j, 16)] = zeros16

    @pl.loop(0, RPT // ZB)
    def _(i):
        pltpu.sync_copy(zb, acc.at[pl.ds(s * RPT + i * ZB, ZB)])

    plsc.subcore_barrier()

    base0 = wid * EPT
    iot = lax.iota(jnp.int32, 16)

    def build(dstv, typv, oh, i):
        base = base0 + i * C
        pltpu.sync_copy(dst_h.at[pl.ds(base, C)], dstv)
        pltpu.sync_copy(typ_h.at[pl.ds(base, C)], typv)

        @pl.loop(0, C, unroll=8)
        def _(e):
            t = plsc.load_gather(typv, [jnp.full((16,), e, jnp.int32)])
            oh[e, pl.ds(0, 16)] = jnp.where(iot == t, 1.0, 0.0).astype(_F32)

    @pl.loop(0, RING, step=NBUF)
    def _(p):
        for b in range(NBUF):
            i = p + b
            dstv, typv, oh = bufs[b]

            @pl.when(i >= NBUF)
            def _():
                pltpu.make_async_copy(oh, acc.at[dstv], ssems[b]).wait()

            build(dstv, typv, oh, i)
            pltpu.async_copy(oh, acc.at[dstv], ssems[b], add=True)

    dstv, typv, oh = bufs[0]
    pltpu.make_async_copy(oh, acc.at[dstv], ssems[0]).wait()
    build(dstv, typv, oh, RING)
    pltpu.sync_copy(oh, acc.at[dstv], add=True)

    for b in range(1, NBUF):
        dstv, typv, oh = bufs[b]
        pltpu.make_async_copy(oh, acc.at[dstv], ssems[b]).wait()

    plsc.subcore_barrier()
    pltpu.sync_copy(acc.at[pl.ds(s * RPT, RPT)],
                    out_h.at[c, pl.ds(s * RPT, RPT)])


def _sc_count(dst, typ):
    per_buf = [pltpu.VMEM((C,), jnp.int32),
               pltpu.VMEM((C,), jnp.int32),
               pltpu.VMEM((C, D), _F32)]
    return pl.kernel(
        _sc_count_body,
        out_type=jax.ShapeDtypeStruct((NC, NPAD, D), _F32),
        mesh=_MESH,
        compiler_params=_SC_PARAMS,
        scratch_types=(
            [pltpu.VMEM((ZB, D), _F32),
             pltpu.VMEM_SHARED((NPAD, D), _F32)]
            + per_buf * NBUF
            + [pltpu.SemaphoreType.DMA] * NBUF
        ),
    )(dst, typ)


def _sc_wg_body(winv_h, src_h, dst_h, typ_h, w_h, g_h, *rest):
    bufs = [rest[6 * b:6 * b + 6] for b in range(NBUF)]
    grsems = rest[6 * NBUF:7 * NBUF]
    wssems = rest[7 * NBUF:8 * NBUF]
    gssems = rest[8 * NBUF:9 * NBUF]
    c, s, wid = _worker_ids()
    base0 = wid * EPT
    iot = lax.iota(jnp.int32, 16)

    def load_meta(b, i):
        srcv, dstv, typv, wrows, wv, gv = bufs[b]
        base = base0 + i * C
        pltpu.sync_copy(src_h.at[pl.ds(base, C)], srcv)
        pltpu.sync_copy(dst_h.at[pl.ds(base, C)], dstv)
        pltpu.sync_copy(typ_h.at[pl.ds(base, C)], typv)

    for b in range(NBUF):
        load_meta(b, b)
        srcv, dstv, typv, wrows, wv, gv = bufs[b]
        pltpu.async_copy(winv_h.at[dstv], wrows, grsems[b])

    def extract(srcv, typv, wrows, wv, gv):
        for k in range(C // 16):
            sl = pl.ds(16 * k, 16)
            tt = typv[sl]
            wv[sl] = plsc.load_gather(wrows, [iot + 16 * k, tt])
            gv[sl] = tt * N + srcv[sl]

    @pl.loop(0, RING, step=NBUF)
    def _(p):
        for b in range(NBUF):
            i = p + b
            srcv, dstv, typv, wrows, wv, gv = bufs[b]
            base = base0 + i * C
            pltpu.make_async_copy(winv_h.at[dstv], wrows, grsems[b]).wait()
            extract(srcv, typv, wrows, wv, gv)
            pltpu.async_copy(wv, w_h.at[pl.ds(base, C)], wssems[b])
            pltpu.async_copy(gv, g_h.at[pl.ds(base, C)], gssems[b])

            @pl.when(i + NBUF < RING)
            def _():
                pltpu.make_async_copy(
                    wv, w_h.at[pl.ds(base, C)], wssems[b]).wait()
                pltpu.make_async_copy(
                    gv, g_h.at[pl.ds(base, C)], gssems[b]).wait()
                load_meta(b, i + NBUF)
                pltpu.async_copy(winv_h.at[dstv], wrows, grsems[b])

    srcv, dstv, typv, wrows, wv, gv = bufs[0]
    base = base0 + RING * C
    pltpu.make_async_copy(wv, w_h.at[pl.ds(base, C)], wssems[0]).wait()
    pltpu.make_async_copy(gv, g_h.at[pl.ds(base, C)], gssems[0]).wait()
    load_meta(0, RING)
    pltpu.async_copy(winv_h.at[dstv], wrows, grsems[0]).wait()
    extract(srcv, typv, wrows, wv, gv)
    pltpu.sync_copy(wv, w_h.at[pl.ds(base, C)])
    pltpu.sync_copy(gv, g_h.at[pl.ds(base, C)])

    for b in range(1, NBUF):
        srcv, dstv, typv, wrows, wv, gv = bufs[b]
        base = base0 + (RING - NBUF + b) * C
        pltpu.make_async_copy(wv, w_h.at[pl.ds(base, C)], wssems[b]).wait()
        pltpu.make_async_copy(gv, g_h.at[pl.ds(base, C)], gssems[b]).wait()


def _sc_wg(winv, src, dst, typ):
    per_buf = [pltpu.VMEM((C,), jnp.int32),
               pltpu.VMEM((C,), jnp.int32),
               pltpu.VMEM((C,), jnp.int32),
               pltpu.VMEM((C, D), _F32),
               pltpu.VMEM((C,), _F32),
               pltpu.VMEM((C,), jnp.int32)]
    return pl.kernel(
        _sc_wg_body,
        out_type=(jax.ShapeDtypeStruct((E,), _F32),
                  jax.ShapeDtypeStruct((E,), jnp.int32)),
        mesh=_MESH,
        compiler_params=_SC_PARAMS,
        scratch_types=(
            per_buf * NBUF
            + [pltpu.SemaphoreType.DMA] * (3 * NBUF)
        ),
    )(winv, src, dst, typ)


def _sc_layer_body(y_h, g_h, w_h, dst_h, out_h, zb, acc, *rest):
    bufs = [rest[4 * b:4 * b + 4] for b in range(NBUF)]
    gsems = rest[4 * NBUF:5 * NBUF]
    ssems = rest[5 * NBUF:6 * NBUF]
    c, s, wid = _worker_ids()
    zeros16 = jnp.zeros((16,), _F32)

    @pl.loop(0, ZB)
    def _(i):
        for j in range(D // 16):
            zb[i, pl.ds(16 * j, 16)] = zeros16

    @pl.loop(0, RPT // ZB)
    def _(i):
        pltpu.sync_copy(zb, acc.at[pl.ds(s * RPT + i * ZB, ZB)])

    plsc.subcore_barrier()

    base0 = wid * EPT

    def load_meta(b, i):
        gv, wv, dstv, yb = bufs[b]
        base = base0 + i * C
        pltpu.sync_copy(g_h.at[pl.ds(base, C)], gv)
        pltpu.sync_copy(w_h.at[pl.ds(base, C)], wv)
        pltpu.sync_copy(dst_h.at[pl.ds(base, C)], dstv)

    for b in range(NBUF):
        load_meta(b, b)
        gv, wv, dstv, yb = bufs[b]
        pltpu.async_copy(y_h.at[gv], yb, gsems[b])

    def scale(wv, yb):
        @pl.loop(0, C, unroll=8)
        def _(e):
            we = plsc.load_gather(wv, [jnp.full((16,), e, jnp.int32)])
            for j in range(D // 16):
                sl = pl.ds(16 * j, 16)
                yb[e, sl] = yb[e, sl] * we

    @pl.loop(0, RING, step=NBUF)
    def _(p):
        for b in range(NBUF):
            i = p + b
            gv, wv, dstv, yb = bufs[b]
            pltpu.make_async_copy(y_h.at[gv], yb, gsems[b]).wait()
            scale(wv, yb)
            pltpu.async_copy(yb, acc.at[dstv], ssems[b], add=True)

            @pl.when(i + NBUF < RING)
            def _():
                pltpu.make_async_copy(yb, acc.at[dstv], ssems[b]).wait()
                load_meta(b, i + NBUF)
                pltpu.async_copy(y_h.at[gv], yb, gsems[b])

    gv, wv, dstv, yb = bufs[0]
    pltpu.make_async_copy(yb, acc.at[dstv], ssems[0]).wait()
    load_meta(0, RING)
    pltpu.async_copy(y_h.at[gv], yb, gsems[0]).wait()
    scale(wv, yb)
    pltpu.sync_copy(yb, acc.at[dstv], add=True)

    for b in range(1, NBUF):
        gv, wv, dstv, yb = bufs[b]
        pltpu.make_async_copy(yb, acc.at[dstv], ssems[b]).wait()

    plsc.subcore_barrier()
    pltpu.sync_copy(acc.at[pl.ds(s * RPT, RPT)],
                    out_h.at[c, pl.ds(s * RPT, RPT)])


def _sc_layer(yflat, g, w, dst):
    per_buf = [pltpu.VMEM((C,), jnp.int32),
               pltpu.VMEM((C,), _F32),
               pltpu.VMEM((C,), jnp.int32),
               pltpu.VMEM((C, D), _F32)]
    return pl.kernel(
        _sc_layer_body,
        out_type=jax.ShapeDtypeStruct((NC, NPAD, D), _F32),
        mesh=_MESH,
        compiler_params=_SC_PARAMS,
        scratch_types=(
            [pltpu.VMEM((ZB, D), _F32),
             pltpu.VMEM_SHARED((NPAD, D), _F32)]
            + per_buf * NBUF
            + [pltpu.SemaphoreType.DMA] * (2 * NBUF)
        ),
    )(yflat, g, w, dst)


def _tc_weights_body(comp_ref, basis_ref, out_ref):
    out_ref[0] = jnp.dot(comp_ref[0], basis_ref[0],
                         preferred_element_type=_F32)


def _tc_weights(comp_all, basis_all):
    return pl.pallas_call(
        _tc_weights_body,
        grid=(3,),
        in_specs=[
            pl.BlockSpec((1, R, NB), lambda i: (i, 0, 0)),
            pl.BlockSpec((1, NB, D * D), lambda i: (i, 0, 0)),
        ],
        out_specs=pl.BlockSpec((1, R, D * D), lambda i: (i, 0, 0)),
        out_shape=jax.ShapeDtypeStruct((3, R, D * D), _F32),
    )(comp_all, basis_all)


def _tc_mm_body(h_ref, w_ref, y_ref):
    h = h_ref[...]
    for r in range(R):
        y_ref[r] = jnp.dot(h, w_ref[r], preferred_element_type=_F32)


def _tc_mm(h, w3):
    return pl.pallas_call(
        _tc_mm_body,
        grid=(NBLK,),
        in_specs=[
            pl.BlockSpec((BN, D), lambda j: (j, 0)),
            pl.BlockSpec((R, D, D), lambda j: (0, 0, 0)),
        ],
        out_specs=pl.BlockSpec((R, BN, D), lambda j: (0, j, 0)),
        out_shape=jax.ShapeDtypeStruct((R, N, D), _F32),
    )(h, w3)


def _tc_winv_body(cnt_ref, out_ref):
    csum = cnt_ref[0] + cnt_ref[1]
    out_ref[...] = 1.0 / jnp.maximum(csum, 1.0)


def _tc_winv(cnts):
    return pl.pallas_call(
        _tc_winv_body,
        grid=(NS,),
        in_specs=[pl.BlockSpec((NC, RPT, D), lambda j: (0, j, 0))],
        out_specs=pl.BlockSpec((RPT, D), lambda j: (j, 0)),
        out_shape=jax.ShapeDtypeStruct((NPAD, D), _F32),
    )(cnts)


def _tc_dense_body(h_ref, root_ref, bias_ref, acc_ref, out_ref, *, relu):
    o = jnp.dot(h_ref[...], root_ref[...], preferred_element_type=_F32)
    o = o + bias_ref[...] + acc_ref[0] + acc_ref[1]
    if relu:
        o = jnp.maximum(o, 0.0)
    out_ref[...] = o


def _tc_dense(h, root, bias, acc, relu):
    return pl.pallas_call(
        functools.partial(_tc_dense_body, relu=relu),
        grid=(NBLK,),
        in_specs=[
            pl.BlockSpec((BN, D), lambda j: (j, 0)),
            pl.BlockSpec((D, D), lambda j: (0, 0)),
            pl.BlockSpec((1, D), lambda j: (0, 0)),
            pl.BlockSpec((NC, BN, D), lambda j: (0, j, 0)),
        ],
        out_specs=pl.BlockSpec((BN, D), lambda j: (j, 0)),
        out_shape=jax.ShapeDtypeStruct((N, D), _F32),
    )(h, root, bias, acc)


def kernel(x, edge_index, edge_type,
           comp1, basis1, root1, bias1,
           comp2, basis2, root2, bias2,
           comp3, basis3, root3, bias3):
    src = edge_index[0]
    dst = edge_index[1]

    comp_all = jnp.stack([comp1, comp2, comp3])
    basis_all = jnp.stack([basis1.reshape(NB, D * D),
                           basis2.reshape(NB, D * D),
                           basis3.reshape(NB, D * D)])
    w_all = _tc_weights(comp_all, basis_all).reshape(3, R, D, D)

    cnts = _sc_count(dst, edge_type)
    winv = _tc_winv(cnts)
    w_e, g_e = _sc_wg(winv, src, dst, edge_type)

    roots = (root1, root2, root3)
    biases = (bias1, bias2, bias3)
    h = x
    for i in range(3):
        y = _tc_mm(h, w_all[i])
        acc = _sc_layer(y.reshape(R * N, D), g_e, w_e, dst)
        h = _tc_dense(h, roots[i], biases[i].reshape(1, D), acc,
                      relu=(i < 2))
    return h


# speedup vs baseline: 47.8309x; 1.5708x over previous
import functools

import jax
import jax.numpy as jnp
from jax import lax
from jax.experimental import pallas as pl
from jax.experimental.pallas import tpu as pltpu
from jax.experimental.pallas import tpu_sc as plsc

N = 10000
E = 320000
D = 128
R = 16
NB = 8

NC = 2
NS = 16
NW = NC * NS
EPT = E // NW
C = 80
NCH = EPT // C
NBUF = 4
RING = NCH - 1
NPAD = 10240
RPT = NPAD // NS
ZB = 16

BN = 400
NBLK = N // BN

_MESH = plsc.VectorSubcoreMesh(core_axis_name="c", subcore_axis_name="s")
_SC_PARAMS = pltpu.CompilerParams(needs_layout_passes=False)
_F32 = jnp.float32


def _worker_ids():
    c = lax.axis_index("c")
    s = lax.axis_index("s")
    return c, s, s * NC + c


def _sc_count_body(dst_h, typ_h, out_h, zb, acc, *rest):
    bufs = [rest[3 * b:3 * b + 3] for b in range(NBUF)]
    ssems = rest[3 * NBUF:4 * NBUF]
    c, s, wid = _worker_ids()
    zeros16 = jnp.zeros((16,), _F32)

    @pl.loop(0, ZB)
    def _(i):
        for j in range(D // 16):
            zb[i, pl.ds(16 * j, 16)] = zeros16

    for b in range(NBUF):
        _, _, oh = bufs[b]

        @pl.loop(0, C)
        def _(e):
            for j in range(D // 16):
                oh[e, pl.ds(16 * j, 16)] = zeros16

    @pl.loop(0, RPT // ZB)
    def _(i):
        pltpu.sync_copy(zb, acc.at[pl.ds(s * RPT + i * ZB, ZB)])

    plsc.subcore_barrier()

    base0 = wid * EPT
    iot = lax.iota(jnp.int32, 16)

    def build(dstv, typv, oh, i):
        base = base0 + i * C
        pltpu.sync_copy(dst_h.at[pl.ds(base, C)], dstv)
        pltpu.sync_copy(typ_h.at[pl.ds(base, C)], typv)

        @pl.loop(0, C, unroll=8)
        def _(e):
            t = plsc.load_gather(typv, [jnp.full((16,), e, jnp.int32)])
            oh[e, pl.ds(0, 16)] = jnp.where(iot == t, 1.0, 0.0).astype(_F32)

    @pl.loop(0, RING, step=NBUF)
    def _(p):
        for b in range(NBUF):
            i = p + b
            dstv, typv, oh = bufs[b]

            @pl.when(i >= NBUF)
            def _():
                pltpu.make_async_copy(oh, acc.at[dstv], ssems[b]).wait()

            build(dstv, typv, oh, i)
            pltpu.async_copy(oh, acc.at[dstv], ssems[b], add=True)

    dstv, typv, oh = bufs[0]
    pltpu.make_async_copy(oh, acc.at[dstv], ssems[0]).wait()
    build(dstv, typv, oh, RING)
    pltpu.sync_copy(oh, acc.at[dstv], add=True)

    for b in range(1, NBUF):
        dstv, typv, oh = bufs[b]
        pltpu.make_async_copy(oh, acc.at[dstv], ssems[b]).wait()

    plsc.subcore_barrier()
    pltpu.sync_copy(acc.at[pl.ds(s * RPT, RPT)],
                    out_h.at[c, pl.ds(s * RPT, RPT)])


def _sc_count(dst, typ):
    per_buf = [pltpu.VMEM((C,), jnp.int32),
               pltpu.VMEM((C,), jnp.int32),
               pltpu.VMEM((C, D), _F32)]
    return pl.kernel(
        _sc_count_body,
        out_type=jax.ShapeDtypeStruct((NC, NPAD, D), _F32),
        mesh=_MESH,
        compiler_params=_SC_PARAMS,
        scratch_types=(
            [pltpu.VMEM((ZB, D), _F32),
             pltpu.VMEM_SHARED((NPAD, D), _F32)]
            + per_buf * NBUF
            + [pltpu.SemaphoreType.DMA] * NBUF
        ),
    )(dst, typ)


def _sc_wg_body(winv_h, src_h, dst_h, typ_h, w_h, g_h, *rest):
    bufs = [rest[6 * b:6 * b + 6] for b in range(NBUF)]
    grsems = rest[6 * NBUF:7 * NBUF]
    wssems = rest[7 * NBUF:8 * NBUF]
    gssems = rest[8 * NBUF:9 * NBUF]
    c, s, wid = _worker_ids()
    base0 = wid * EPT
    iot = lax.iota(jnp.int32, 16)

    def load_meta(b, i):
        srcv, dstv, typv, wrows, wv, gv = bufs[b]
        base = base0 + i * C
        pltpu.sync_copy(src_h.at[pl.ds(base, C)], srcv)
        pltpu.sync_copy(dst_h.at[pl.ds(base, C)], dstv)
        pltpu.sync_copy(typ_h.at[pl.ds(base, C)], typv)

    for b in range(NBUF):
        load_meta(b, b)
        srcv, dstv, typv, wrows, wv, gv = bufs[b]
        pltpu.async_copy(winv_h.at[dstv], wrows, grsems[b])

    def extract(srcv, typv, wrows, wv, gv):
        for k in range(C // 16):
            sl = pl.ds(16 * k, 16)
            tt = typv[sl]
            wv[sl] = plsc.load_gather(wrows, [iot + 16 * k, tt])
            gv[sl] = tt * N + srcv[sl]

    @pl.loop(0, RING, step=NBUF)
    def _(p):
        for b in range(NBUF):
            i = p + b
            srcv, dstv, typv, wrows, wv, gv = bufs[b]
            base = base0 + i * C
            pltpu.make_async_copy(winv_h.at[dstv], wrows, grsems[b]).wait()
            extract(srcv, typv, wrows, wv, gv)
            pltpu.async_copy(wv, w_h.at[pl.ds(base, C)], wssems[b])
            pltpu.async_copy(gv, g_h.at[pl.ds(base, C)], gssems[b])

            @pl.when(i + NBUF < RING)
            def _():
                pltpu.make_async_copy(
                    wv, w_h.at[pl.ds(base, C)], wssems[b]).wait()
                pltpu.make_async_copy(
                    gv, g_h.at[pl.ds(base, C)], gssems[b]).wait()
                load_meta(b, i + NBUF)
                pltpu.async_copy(winv_h.at[dstv], wrows, grsems[b])

    srcv, dstv, typv, wrows, wv, gv = bufs[0]
    base = base0 + RING * C
    pltpu.make_async_copy(wv, w_h.at[pl.ds(base, C)], wssems[0]).wait()
    pltpu.make_async_copy(gv, g_h.at[pl.ds(base, C)], gssems[0]).wait()
    load_meta(0, RING)
    pltpu.async_copy(winv_h.at[dstv], wrows, grsems[0]).wait()
    extract(srcv, typv, wrows, wv, gv)
    pltpu.sync_copy(wv, w_h.at[pl.ds(base, C)])
    pltpu.sync_copy(gv, g_h.at[pl.ds(base, C)])

    for b in range(1, NBUF):
        srcv, dstv, typv, wrows, wv, gv = bufs[b]
        base = base0 + (RING - NBUF + b) * C
        pltpu.make_async_copy(wv, w_h.at[pl.ds(base, C)], wssems[b]).wait()
        pltpu.make_async_copy(gv, g_h.at[pl.ds(base, C)], gssems[b]).wait()


def _sc_wg(winv, src, dst, typ):
    per_buf = [pltpu.VMEM((C,), jnp.int32),
               pltpu.VMEM((C,), jnp.int32),
               pltpu.VMEM((C,), jnp.int32),
               pltpu.VMEM((C, D), _F32),
               pltpu.VMEM((C,), _F32),
               pltpu.VMEM((C,), jnp.int32)]
    return pl.kernel(
        _sc_wg_body,
        out_type=(jax.ShapeDtypeStruct((E,), _F32),
                  jax.ShapeDtypeStruct((E,), jnp.int32)),
        mesh=_MESH,
        compiler_params=_SC_PARAMS,
        scratch_types=(
            per_buf * NBUF
            + [pltpu.SemaphoreType.DMA] * (3 * NBUF)
        ),
    )(winv, src, dst, typ)


def _sc_layer_body(y_h, g_h, w_h, dst_h, out_h, zb, acc, *rest):
    bufs = [rest[4 * b:4 * b + 4] for b in range(NBUF)]
    msems = rest[4 * NBUF:5 * NBUF]
    gsems = rest[5 * NBUF:6 * NBUF]
    ssems = rest[6 * NBUF:7 * NBUF]
    c, s, wid = _worker_ids()
    zeros16 = jnp.zeros((16,), _F32)

    @pl.loop(0, ZB)
    def _(i):
        for j in range(D // 16):
            zb[i, pl.ds(16 * j, 16)] = zeros16

    @pl.loop(0, RPT // ZB)
    def _(i):
        pltpu.sync_copy(zb, acc.at[pl.ds(s * RPT + i * ZB, ZB)])

    plsc.subcore_barrier()

    base0 = wid * EPT

    def issue_meta(b, i):
        gv, wv, dstv, yb = bufs[b]
        base = base0 + i * C
        pltpu.async_copy(g_h.at[pl.ds(base, C)], gv, msems[b])
        pltpu.async_copy(w_h.at[pl.ds(base, C)], wv, msems[b])
        pltpu.async_copy(dst_h.at[pl.ds(base, C)], dstv, msems[b])

    def wait_meta(b, i):
        gv, wv, dstv, yb = bufs[b]
        base = base0 + i * C
        pltpu.make_async_copy(g_h.at[pl.ds(base, C)], gv, msems[b]).wait()
        pltpu.make_async_copy(w_h.at[pl.ds(base, C)], wv, msems[b]).wait()
        pltpu.make_async_copy(dst_h.at[pl.ds(base, C)], dstv, msems[b]).wait()

    def scale(wv, yb):
        @pl.loop(0, C, unroll=8)
        def _(e):
            we = plsc.load_gather(wv, [jnp.full((16,), e, jnp.int32)])
            for j in range(D // 16):
                sl = pl.ds(16 * j, 16)
                yb[e, sl] = yb[e, sl] * we

    for t in range(3):
        issue_meta(t % NBUF, t)
    for t in range(2):
        b = t % NBUF
        gv, _, _, yb = bufs[b]
        wait_meta(b, t)
        pltpu.async_copy(y_h.at[gv], yb, gsems[b])

    nloops = (NCH + NBUF - 1) // NBUF * NBUF

    @pl.loop(0, nloops, step=NBUF)
    def _(p):
        for b in range(NBUF):
            i = p + b
            gv, wv, dstv, yb = bufs[b]

            @pl.when(i < NCH)
            def _():
                pltpu.make_async_copy(y_h.at[gv], yb, gsems[b]).wait()
                scale(wv, yb)
                pltpu.async_copy(yb, acc.at[dstv], ssems[b], add=True)

            t3 = i + 3
            b3 = (b + 3) % NBUF
            gv3, wv3, dstv3, yb3 = bufs[b3]

            @pl.when(t3 < NCH)
            def _():
                @pl.when(t3 >= NBUF)
                def _():
                    pltpu.make_async_copy(
                        yb3, acc.at[dstv3], ssems[b3]).wait()

                issue_meta(b3, t3)

            t2 = i + 2
            b2 = (b + 2) % NBUF
            gv2, wv2, dstv2, yb2 = bufs[b2]

            @pl.when(t2 < NCH)
            def _():
                wait_meta(b2, t2)
                pltpu.async_copy(y_h.at[gv2], yb2, gsems[b2])

    for b in range(NBUF):
        gv, wv, dstv, yb = bufs[b]
        pltpu.make_async_copy(yb, acc.at[dstv], ssems[b]).wait()

    plsc.subcore_barrier()
    pltpu.sync_copy(acc.at[pl.ds(s * RPT, RPT)],
                    out_h.at[c, pl.ds(s * RPT, RPT)])


def _sc_layer(yflat, g, w, dst):
    per_buf = [pltpu.VMEM((C,), jnp.int32),
               pltpu.VMEM((C,), _F32),
               pltpu.VMEM((C,), jnp.int32),
               pltpu.VMEM((C, D), _F32)]
    return pl.kernel(
        _sc_layer_body,
        out_type=jax.ShapeDtypeStruct((NC, NPAD, D), _F32),
        mesh=_MESH,
        compiler_params=_SC_PARAMS,
        scratch_types=(
            [pltpu.VMEM((ZB, D), _F32),
             pltpu.VMEM_SHARED((NPAD, D), _F32)]
            + per_buf * NBUF
            + [pltpu.SemaphoreType.DMA] * (3 * NBUF)
        ),
    )(yflat, g, w, dst)


def _tc_weights_body(comp_ref, basis_ref, out_ref):
    out_ref[0] = jnp.dot(comp_ref[0], basis_ref[0],
                         preferred_element_type=_F32)


def _tc_weights(comp_all, basis_all):
    return pl.pallas_call(
        _tc_weights_body,
        grid=(3,),
        in_specs=[
            pl.BlockSpec((1, R, NB), lambda i: (i, 0, 0)),
            pl.BlockSpec((1, NB, D * D), lambda i: (i, 0, 0)),
        ],
        out_specs=pl.BlockSpec((1, R, D * D), lambda i: (i, 0, 0)),
        out_shape=jax.ShapeDtypeStruct((3, R, D * D), _F32),
    )(comp_all, basis_all)


def _tc_mm_body(h_ref, w_ref, y_ref):
    h = h_ref[...]
    for r in range(R):
        y_ref[r] = jnp.dot(h, w_ref[r], preferred_element_type=_F32)


def _tc_mm(h, w3):
    return pl.pallas_call(
        _tc_mm_body,
        grid=(NBLK,),
        in_specs=[
            pl.BlockSpec((BN, D), lambda j: (j, 0)),
            pl.BlockSpec((R, D, D), lambda j: (0, 0, 0)),
        ],
        out_specs=pl.BlockSpec((R, BN, D), lambda j: (0, j, 0)),
        out_shape=jax.ShapeDtypeStruct((R, N, D), _F32),
    )(h, w3)


def _tc_winv_body(cnt_ref, out_ref):
    csum = cnt_ref[0] + cnt_ref[1]
    out_ref[...] = 1.0 / jnp.maximum(csum, 1.0)


def _tc_winv(cnts):
    return pl.pallas_call(
        _tc_winv_body,
        grid=(NS,),
        in_specs=[pl.BlockSpec((NC, RPT, D), lambda j: (0, j, 0))],
        out_specs=pl.BlockSpec((RPT, D), lambda j: (j, 0)),
        out_shape=jax.ShapeDtypeStruct((NPAD, D), _F32),
    )(cnts)


def _tc_dense_body(h_ref, root_ref, bias_ref, acc_ref, out_ref, *, relu):
    o = jnp.dot(h_ref[...], root_ref[...], preferred_element_type=_F32)
    o = o + bias_ref[...] + acc_ref[0] + acc_ref[1]
    if relu:
        o = jnp.maximum(o, 0.0)
    out_ref[...] = o


def _tc_dense(h, root, bias, acc, relu):
    return pl.pallas_call(
        functools.partial(_tc_dense_body, relu=relu),
        grid=(NBLK,),
        in_specs=[
            pl.BlockSpec((BN, D), lambda j: (j, 0)),
            pl.BlockSpec((D, D), lambda j: (0, 0)),
            pl.BlockSpec((1, D), lambda j: (0, 0)),
            pl.BlockSpec((NC, BN, D), lambda j: (0, j, 0)),
        ],
        out_specs=pl.BlockSpec((BN, D), lambda j: (j, 0)),
        out_shape=jax.ShapeDtypeStruct((N, D), _F32),
    )(h, root, bias, acc)


def kernel(x, edge_index, edge_type,
           comp1, basis1, root1, bias1,
           comp2, basis2, root2, bias2,
           comp3, basis3, root3, bias3):
    src = edge_index[0]
    dst = edge_index[1]

    comp_all = jnp.stack([comp1, comp2, comp3])
    basis_all = jnp.stack([basis1.reshape(NB, D * D),
                           basis2.reshape(NB, D * D),
                           basis3.reshape(NB, D * D)])
    w_all = _tc_weights(comp_all, basis_all).reshape(3, R, D, D)

    cnts = _sc_count(dst, edge_type)
    winv = _tc_winv(cnts)
    w_e, g_e = _sc_wg(winv, src, dst, edge_type)

    roots = (root1, root2, root3)
    biases = (bias1, bias2, bias3)
    h = x
    for i in range(3):
        y = _tc_mm(h, w_all[i])
        acc = _sc_layer(y.reshape(R * N, D), g_e, w_e, dst)
        h = _tc_dense(h, roots[i], biases[i].reshape(1, D), acc,
                      relu=(i < 2))
    return h


# speedup vs baseline: 59.7766x; 1.2497x over previous
import functools

import jax
import jax.numpy as jnp
from jax import lax
from jax.experimental import pallas as pl
from jax.experimental.pallas import tpu as pltpu
from jax.experimental.pallas import tpu_sc as plsc

N = 10000
E = 320000
D = 128
R = 16
NB = 8

NC = 2
NS = 16
NW = NC * NS
EPT = E // NW
C = 80
NCH = EPT // C
NBUF = 4
RING = NCH - 1
NPAD = 10240
RPT = NPAD // NS
ZB = 16

BN = 400
NBLK = N // BN

_MESH = plsc.VectorSubcoreMesh(core_axis_name="c", subcore_axis_name="s")
_SC_PARAMS = pltpu.CompilerParams(needs_layout_passes=False)
_F32 = jnp.float32


def _worker_ids():
    c = lax.axis_index("c")
    s = lax.axis_index("s")
    return c, s, s * NC + c


def _sc_count_body(dst_h, typ_h, out_h, zb, acc, *rest):
    bufs = [rest[3 * b:3 * b + 3] for b in range(NBUF)]
    msems = rest[3 * NBUF:4 * NBUF]
    ssems = rest[4 * NBUF:5 * NBUF]
    c, s, wid = _worker_ids()
    zeros16 = jnp.zeros((16,), _F32)

    @pl.loop(0, ZB)
    def _(i):
        for j in range(D // 16):
            zb[i, pl.ds(16 * j, 16)] = zeros16

    for b in range(NBUF):
        _, _, oh = bufs[b]

        @pl.loop(0, C)
        def _(e):
            for j in range(D // 16):
                oh[e, pl.ds(16 * j, 16)] = zeros16

    @pl.loop(0, RPT // ZB)
    def _(i):
        pltpu.sync_copy(zb, acc.at[pl.ds(s * RPT + i * ZB, ZB)])

    plsc.subcore_barrier()

    base0 = wid * EPT
    iot = lax.iota(jnp.int32, 16)

    def issue_meta(b, i):
        dstv, typv, oh = bufs[b]
        base = base0 + i * C
        pltpu.async_copy(dst_h.at[pl.ds(base, C)], dstv, msems[b])
        pltpu.async_copy(typ_h.at[pl.ds(base, C)], typv, msems[b])

    def wait_meta(b, i):
        dstv, typv, oh = bufs[b]
        base = base0 + i * C
        pltpu.make_async_copy(dst_h.at[pl.ds(base, C)], dstv, msems[b]).wait()
        pltpu.make_async_copy(typ_h.at[pl.ds(base, C)], typv, msems[b]).wait()

    for t in range(3):
        issue_meta(t % NBUF, t)

    nloops = (NCH + NBUF - 1) // NBUF * NBUF

    @pl.loop(0, nloops, step=NBUF)
    def _(p):
        for b in range(NBUF):
            i = p + b
            dstv, typv, oh = bufs[b]

            @pl.when(i < NCH)
            def _():
                wait_meta(b, i)

                @pl.loop(0, C, unroll=8)
                def _(e):
                    t = plsc.load_gather(
                        typv, [jnp.full((16,), e, jnp.int32)])
                    oh[e, pl.ds(0, 16)] = jnp.where(
                        iot == t, 1.0, 0.0).astype(_F32)

                pltpu.async_copy(oh, acc.at[dstv], ssems[b], add=True)

            t3 = i + 3
            b3 = (b + 3) % NBUF
            dstv3, typv3, oh3 = bufs[b3]

            @pl.when(t3 < NCH)
            def _():
                @pl.when(t3 >= NBUF)
                def _():
                    pltpu.make_async_copy(
                        oh3, acc.at[dstv3], ssems[b3]).wait()

                issue_meta(b3, t3)

    for b in range(NBUF):
        dstv, typv, oh = bufs[b]
        pltpu.make_async_copy(oh, acc.at[dstv], ssems[b]).wait()

    plsc.subcore_barrier()
    pltpu.sync_copy(acc.at[pl.ds(s * RPT, RPT)],
                    out_h.at[c, pl.ds(s * RPT, RPT)])


def _sc_count(dst, typ):
    per_buf = [pltpu.VMEM((C,), jnp.int32),
               pltpu.VMEM((C,), jnp.int32),
               pltpu.VMEM((C, D), _F32)]
    return pl.kernel(
        _sc_count_body,
        out_type=jax.ShapeDtypeStruct((NC, NPAD, D), _F32),
        mesh=_MESH,
        compiler_params=_SC_PARAMS,
        scratch_types=(
            [pltpu.VMEM((ZB, D), _F32),
             pltpu.VMEM_SHARED((NPAD, D), _F32)]
            + per_buf * NBUF
            + [pltpu.SemaphoreType.DMA] * (2 * NBUF)
        ),
    )(dst, typ)


def _sc_wg_body(winv_h, src_h, dst_h, typ_h, w_h, g_h, *rest):
    bufs = [rest[6 * b:6 * b + 6] for b in range(NBUF)]
    msems = rest[6 * NBUF:7 * NBUF]
    grsems = rest[7 * NBUF:8 * NBUF]
    osems = rest[8 * NBUF:9 * NBUF]
    c, s, wid = _worker_ids()
    base0 = wid * EPT
    iot = lax.iota(jnp.int32, 16)

    def issue_meta(b, i):
        srcv, dstv, typv, wrows, wv, gv = bufs[b]
        base = base0 + i * C
        pltpu.async_copy(src_h.at[pl.ds(base, C)], srcv, msems[b])
        pltpu.async_copy(dst_h.at[pl.ds(base, C)], dstv, msems[b])
        pltpu.async_copy(typ_h.at[pl.ds(base, C)], typv, msems[b])

    def wait_meta(b, i):
        srcv, dstv, typv, wrows, wv, gv = bufs[b]
        base = base0 + i * C
        pltpu.make_async_copy(src_h.at[pl.ds(base, C)], srcv, msems[b]).wait()
        pltpu.make_async_copy(dst_h.at[pl.ds(base, C)], dstv, msems[b]).wait()
        pltpu.make_async_copy(typ_h.at[pl.ds(base, C)], typv, msems[b]).wait()

    def wait_out(b, i):
        srcv, dstv, typv, wrows, wv, gv = bufs[b]
        base = base0 + i * C
        pltpu.make_async_copy(wv, w_h.at[pl.ds(base, C)], osems[b]).wait()
        pltpu.make_async_copy(gv, g_h.at[pl.ds(base, C)], osems[b]).wait()

    for t in range(3):
        issue_meta(t % NBUF, t)
    for t in range(2):
        b = t % NBUF
        srcv, dstv, typv, wrows, wv, gv = bufs[b]
        wait_meta(b, t)
        pltpu.async_copy(winv_h.at[dstv], wrows, grsems[b])

    nloops = (NCH + NBUF - 1) // NBUF * NBUF

    @pl.loop(0, nloops, step=NBUF)
    def _(p):
        for b in range(NBUF):
            i = p + b
            srcv, dstv, typv, wrows, wv, gv = bufs[b]
            base = base0 + i * C

            @pl.when(i < NCH)
            def _():
                pltpu.make_async_copy(
                    winv_h.at[dstv], wrows, grsems[b]).wait()
                for k in range(C // 16):
                    sl = pl.ds(16 * k, 16)
                    tt = typv[sl]
                    wv[sl] = plsc.load_gather(wrows, [iot + 16 * k, tt])
                    gv[sl] = tt * N + srcv[sl]
                pltpu.async_copy(wv, w_h.at[pl.ds(base, C)], osems[b])
                pltpu.async_copy(gv, g_h.at[pl.ds(base, C)], osems[b])

            t3 = i + 3
            b3 = (b + 3) % NBUF

            @pl.when(t3 < NCH)
            def _():
                @pl.when(t3 >= NBUF)
                def _():
                    wait_out(b3, t3 - NBUF)

                issue_meta(b3, t3)

            t2 = i + 2
            b2 = (b + 2) % NBUF
            srcv2, dstv2, typv2, wrows2, wv2, gv2 = bufs[b2]

            @pl.when(t2 < NCH)
            def _():
                wait_meta(b2, t2)
                pltpu.async_copy(winv_h.at[dstv2], wrows2, grsems[b2])

    for b in range(NBUF):
        wait_out(b, 0)


def _sc_wg(winv, src, dst, typ):
    per_buf = [pltpu.VMEM((C,), jnp.int32),
               pltpu.VMEM((C,), jnp.int32),
               pltpu.VMEM((C,), jnp.int32),
               pltpu.VMEM((C, D), _F32),
               pltpu.VMEM((C,), _F32),
               pltpu.VMEM((C,), jnp.int32)]
    return pl.kernel(
        _sc_wg_body,
        out_type=(jax.ShapeDtypeStruct((E,), _F32),
                  jax.ShapeDtypeStruct((E,), jnp.int32)),
        mesh=_MESH,
        compiler_params=_SC_PARAMS,
        scratch_types=(
            per_buf * NBUF
            + [pltpu.SemaphoreType.DMA] * (3 * NBUF)
        ),
    )(winv, src, dst, typ)


def _sc_layer_body(y_h, g_h, w_h, dst_h, out_h, zb, acc, *rest):
    bufs = [rest[4 * b:4 * b + 4] for b in range(NBUF)]
    msems = rest[4 * NBUF:5 * NBUF]
    gsems = rest[5 * NBUF:6 * NBUF]
    ssems = rest[6 * NBUF:7 * NBUF]
    c, s, wid = _worker_ids()
    zeros16 = jnp.zeros((16,), _F32)

    @pl.loop(0, ZB)
    def _(i):
        for j in range(D // 16):
            zb[i, pl.ds(16 * j, 16)] = zeros16

    @pl.loop(0, RPT // ZB)
    def _(i):
        pltpu.sync_copy(zb, acc.at[pl.ds(s * RPT + i * ZB, ZB)])

    plsc.subcore_barrier()

    base0 = wid * EPT

    def issue_meta(b, i):
        gv, wv, dstv, yb = bufs[b]
        base = base0 + i * C
        pltpu.async_copy(g_h.at[pl.ds(base, C)], gv, msems[b])
        pltpu.async_copy(w_h.at[pl.ds(base, C)], wv, msems[b])
        pltpu.async_copy(dst_h.at[pl.ds(base, C)], dstv, msems[b])

    def wait_meta(b, i):
        gv, wv, dstv, yb = bufs[b]
        base = base0 + i * C
        pltpu.make_async_copy(g_h.at[pl.ds(base, C)], gv, msems[b]).wait()
        pltpu.make_async_copy(w_h.at[pl.ds(base, C)], wv, msems[b]).wait()
        pltpu.make_async_copy(dst_h.at[pl.ds(base, C)], dstv, msems[b]).wait()

    def scale(wv, yb):
        @pl.loop(0, C, unroll=8)
        def _(e):
            we = plsc.load_gather(wv, [jnp.full((16,), e, jnp.int32)])
            for j in range(D // 16):
                sl = pl.ds(16 * j, 16)
                yb[e, sl] = yb[e, sl] * we

    for t in range(3):
        issue_meta(t % NBUF, t)
    for t in range(2):
        b = t % NBUF
        gv, _, _, yb = bufs[b]
        wait_meta(b, t)
        pltpu.async_copy(y_h.at[gv], yb, gsems[b])

    nloops = (NCH + NBUF - 1) // NBUF * NBUF

    @pl.loop(0, nloops, step=NBUF)
    def _(p):
        for b in range(NBUF):
            i = p + b
            gv, wv, dstv, yb = bufs[b]

            @pl.when(i < NCH)
            def _():
                pltpu.make_async_copy(y_h.at[gv], yb, gsems[b]).wait()
                scale(wv, yb)
                pltpu.async_copy(yb, acc.at[dstv], ssems[b], add=True)

            t3 = i + 3
            b3 = (b + 3) % NBUF
            gv3, wv3, dstv3, yb3 = bufs[b3]

            @pl.when(t3 < NCH)
            def _():
                @pl.when(t3 >= NBUF)
                def _():
                    pltpu.make_async_copy(
                        yb3, acc.at[dstv3], ssems[b3]).wait()

                issue_meta(b3, t3)

            t2 = i + 2
            b2 = (b + 2) % NBUF
            gv2, wv2, dstv2, yb2 = bufs[b2]

            @pl.when(t2 < NCH)
            def _():
                wait_meta(b2, t2)
                pltpu.async_copy(y_h.at[gv2], yb2, gsems[b2])

    for b in range(NBUF):
        gv, wv, dstv, yb = bufs[b]
        pltpu.make_async_copy(yb, acc.at[dstv], ssems[b]).wait()

    plsc.subcore_barrier()
    pltpu.sync_copy(acc.at[pl.ds(s * RPT, RPT)],
                    out_h.at[c, pl.ds(s * RPT, RPT)])


def _sc_layer(yflat, g, w, dst):
    per_buf = [pltpu.VMEM((C,), jnp.int32),
               pltpu.VMEM((C,), _F32),
               pltpu.VMEM((C,), jnp.int32),
               pltpu.VMEM((C, D), _F32)]
    return pl.kernel(
        _sc_layer_body,
        out_type=jax.ShapeDtypeStruct((NC, NPAD, D), _F32),
        mesh=_MESH,
        compiler_params=_SC_PARAMS,
        scratch_types=(
            [pltpu.VMEM((ZB, D), _F32),
             pltpu.VMEM_SHARED((NPAD, D), _F32)]
            + per_buf * NBUF
            + [pltpu.SemaphoreType.DMA] * (3 * NBUF)
        ),
    )(yflat, g, w, dst)


def _tc_weights_body(comp_ref, basis_ref, out_ref):
    out_ref[0] = jnp.dot(comp_ref[0], basis_ref[0],
                         preferred_element_type=_F32)


def _tc_weights(comp_all, basis_all):
    return pl.pallas_call(
        _tc_weights_body,
        grid=(3,),
        in_specs=[
            pl.BlockSpec((1, R, NB), lambda i: (i, 0, 0)),
            pl.BlockSpec((1, NB, D * D), lambda i: (i, 0, 0)),
        ],
        out_specs=pl.BlockSpec((1, R, D * D), lambda i: (i, 0, 0)),
        out_shape=jax.ShapeDtypeStruct((3, R, D * D), _F32),
    )(comp_all, basis_all)


def _tc_mm_body(h_ref, w_ref, y_ref):
    h = h_ref[...]
    for r in range(R):
        y_ref[r] = jnp.dot(h, w_ref[r], preferred_element_type=_F32)


def _tc_mm(h, w3):
    return pl.pallas_call(
        _tc_mm_body,
        grid=(NBLK,),
        in_specs=[
            pl.BlockSpec((BN, D), lambda j: (j, 0)),
            pl.BlockSpec((R, D, D), lambda j: (0, 0, 0)),
        ],
        out_specs=pl.BlockSpec((R, BN, D), lambda j: (0, j, 0)),
        out_shape=jax.ShapeDtypeStruct((R, N, D), _F32),
    )(h, w3)


def _tc_winv_body(cnt_ref, out_ref):
    csum = cnt_ref[0] + cnt_ref[1]
    out_ref[...] = 1.0 / jnp.maximum(csum, 1.0)


def _tc_winv(cnts):
    return pl.pallas_call(
        _tc_winv_body,
        grid=(NS,),
        in_specs=[pl.BlockSpec((NC, RPT, D), lambda j: (0, j, 0))],
        out_specs=pl.BlockSpec((RPT, D), lambda j: (j, 0)),
        out_shape=jax.ShapeDtypeStruct((NPAD, D), _F32),
    )(cnts)


def _tc_dense_body(h_ref, root_ref, bias_ref, acc_ref, out_ref, *, relu):
    o = jnp.dot(h_ref[...], root_ref[...], preferred_element_type=_F32)
    o = o + bias_ref[...] + acc_ref[0] + acc_ref[1]
    if relu:
        o = jnp.maximum(o, 0.0)
    out_ref[...] = o


def _tc_dense(h, root, bias, acc, relu):
    return pl.pallas_call(
        functools.partial(_tc_dense_body, relu=relu),
        grid=(NBLK,),
        in_specs=[
            pl.BlockSpec((BN, D), lambda j: (j, 0)),
            pl.BlockSpec((D, D), lambda j: (0, 0)),
            pl.BlockSpec((1, D), lambda j: (0, 0)),
            pl.BlockSpec((NC, BN, D), lambda j: (0, j, 0)),
        ],
        out_specs=pl.BlockSpec((BN, D), lambda j: (j, 0)),
        out_shape=jax.ShapeDtypeStruct((N, D), _F32),
    )(h, root, bias, acc)


def kernel(x, edge_index, edge_type,
           comp1, basis1, root1, bias1,
           comp2, basis2, root2, bias2,
           comp3, basis3, root3, bias3):
    src = edge_index[0]
    dst = edge_index[1]

    comp_all = jnp.stack([comp1, comp2, comp3])
    basis_all = jnp.stack([basis1.reshape(NB, D * D),
                           basis2.reshape(NB, D * D),
                           basis3.reshape(NB, D * D)])
    w_all = _tc_weights(comp_all, basis_all).reshape(3, R, D, D)

    cnts = _sc_count(dst, edge_type)
    winv = _tc_winv(cnts)
    w_e, g_e = _sc_wg(winv, src, dst, edge_type)

    roots = (root1, root2, root3)
    biases = (bias1, bias2, bias3)
    h = x
    for i in range(3):
        y = _tc_mm(h, w_all[i])
        acc = _sc_layer(y.reshape(R * N, D), g_e, w_e, dst)
        h = _tc_dense(h, roots[i], biases[i].reshape(1, D), acc,
                      relu=(i < 2))
    return h


# speedup vs baseline: 62.0622x; 1.0382x over previous
import functools

import jax
import jax.numpy as jnp
from jax import lax
from jax.experimental import pallas as pl
from jax.experimental.pallas import tpu as pltpu
from jax.experimental.pallas import tpu_sc as plsc

N = 10000
E = 320000
D = 128
R = 16
NB = 8

NC = 2
NS = 16
NW = NC * NS
EPT = E // NW
C = 80
NCH = EPT // C
NBUF = 4
RING = NCH - 1
NPAD = 10240
RPT = NPAD // NS
ZB = 16

BN = 400
NBLK = N // BN

_MESH = plsc.VectorSubcoreMesh(core_axis_name="c", subcore_axis_name="s")
_SC_PARAMS = pltpu.CompilerParams(needs_layout_passes=False)
_F32 = jnp.float32


def _worker_ids():
    c = lax.axis_index("c")
    s = lax.axis_index("s")
    return c, s, s * NC + c


def _sc_count_body(dst_h, typ_h, out_h, zb, acc, *rest):
    bufs = [rest[3 * b:3 * b + 3] for b in range(NBUF)]
    msems = rest[3 * NBUF:4 * NBUF]
    ssems = rest[4 * NBUF:5 * NBUF]
    c, s, wid = _worker_ids()
    zeros16 = jnp.zeros((16,), _F32)

    @pl.loop(0, ZB)
    def _(i):
        for j in range(D // 16):
            zb[i, pl.ds(16 * j, 16)] = zeros16

    for b in range(NBUF):
        _, _, oh = bufs[b]

        @pl.loop(0, C)
        def _(e):
            for j in range(D // 16):
                oh[e, pl.ds(16 * j, 16)] = zeros16

    @pl.loop(0, RPT // ZB)
    def _(i):
        pltpu.sync_copy(zb, acc.at[pl.ds(s * RPT + i * ZB, ZB)])

    plsc.subcore_barrier()

    base0 = wid * EPT
    iot = lax.iota(jnp.int32, 16)

    def issue_meta(b, i):
        dstv, typv, oh = bufs[b]
        base = base0 + i * C
        pltpu.async_copy(dst_h.at[pl.ds(base, C)], dstv, msems[b])
        pltpu.async_copy(typ_h.at[pl.ds(base, C)], typv, msems[b])

    def wait_meta(b, i):
        dstv, typv, oh = bufs[b]
        base = base0 + i * C
        pltpu.make_async_copy(dst_h.at[pl.ds(base, C)], dstv, msems[b]).wait()
        pltpu.make_async_copy(typ_h.at[pl.ds(base, C)], typv, msems[b]).wait()

    for t in range(3):
        issue_meta(t % NBUF, t)

    nloops = (NCH + NBUF - 1) // NBUF * NBUF

    @pl.loop(0, nloops, step=NBUF)
    def _(p):
        for b in range(NBUF):
            i = p + b
            dstv, typv, oh = bufs[b]

            @pl.when(i < NCH)
            def _():
                wait_meta(b, i)

                @pl.loop(0, C, unroll=8)
                def _(e):
                    t = plsc.load_gather(
                        typv, [jnp.full((16,), e, jnp.int32)])
                    oh[e, pl.ds(0, 16)] = jnp.where(
                        iot == t, 1.0, 0.0).astype(_F32)

                pltpu.async_copy(oh, acc.at[dstv], ssems[b], add=True)

            t3 = i + 3
            b3 = (b + 3) % NBUF
            dstv3, typv3, oh3 = bufs[b3]

            @pl.when(t3 < NCH)
            def _():
                @pl.when(t3 >= NBUF)
                def _():
                    pltpu.make_async_copy(
                        oh3, acc.at[dstv3], ssems[b3]).wait()

                issue_meta(b3, t3)

    for b in range(NBUF):
        dstv, typv, oh = bufs[b]
        pltpu.make_async_copy(oh, acc.at[dstv], ssems[b]).wait()

    plsc.subcore_barrier()
    pltpu.sync_copy(acc.at[pl.ds(s * RPT, RPT)],
                    out_h.at[c, pl.ds(s * RPT, RPT)])


def _sc_count(dst, typ):
    per_buf = [pltpu.VMEM((C,), jnp.int32),
               pltpu.VMEM((C,), jnp.int32),
               pltpu.VMEM((C, D), _F32)]
    return pl.kernel(
        _sc_count_body,
        out_type=jax.ShapeDtypeStruct((NC, NPAD, D), _F32),
        mesh=_MESH,
        compiler_params=_SC_PARAMS,
        scratch_types=(
            [pltpu.VMEM((ZB, D), _F32),
             pltpu.VMEM_SHARED((NPAD, D), _F32)]
            + per_buf * NBUF
            + [pltpu.SemaphoreType.DMA] * (2 * NBUF)
        ),
    )(dst, typ)


def _sc_wg_body(winv_h, src_h, dst_h, typ_h, w_h, g_h, *rest):
    bufs = [rest[6 * b:6 * b + 6] for b in range(NBUF)]
    msems = rest[6 * NBUF:7 * NBUF]
    grsems = rest[7 * NBUF:8 * NBUF]
    osems = rest[8 * NBUF:9 * NBUF]
    c, s, wid = _worker_ids()
    base0 = wid * EPT
    iot = lax.iota(jnp.int32, 16)

    def issue_meta(b, i):
        srcv, dstv, typv, wrows, wv, gv = bufs[b]
        base = base0 + i * C
        pltpu.async_copy(src_h.at[pl.ds(base, C)], srcv, msems[b])
        pltpu.async_copy(dst_h.at[pl.ds(base, C)], dstv, msems[b])
        pltpu.async_copy(typ_h.at[pl.ds(base, C)], typv, msems[b])

    def wait_meta(b, i):
        srcv, dstv, typv, wrows, wv, gv = bufs[b]
        base = base0 + i * C
        pltpu.make_async_copy(src_h.at[pl.ds(base, C)], srcv, msems[b]).wait()
        pltpu.make_async_copy(dst_h.at[pl.ds(base, C)], dstv, msems[b]).wait()
        pltpu.make_async_copy(typ_h.at[pl.ds(base, C)], typv, msems[b]).wait()

    def wait_out(b, i):
        srcv, dstv, typv, wrows, wv, gv = bufs[b]
        base = base0 + i * C
        pltpu.make_async_copy(wv, w_h.at[pl.ds(base, C)], osems[b]).wait()
        pltpu.make_async_copy(gv, g_h.at[pl.ds(base, C)], osems[b]).wait()

    for t in range(3):
        issue_meta(t % NBUF, t)
    for t in range(2):
        b = t % NBUF
        srcv, dstv, typv, wrows, wv, gv = bufs[b]
        wait_meta(b, t)
        pltpu.async_copy(winv_h.at[dstv], wrows, grsems[b])

    nloops = (NCH + NBUF - 1) // NBUF * NBUF

    @pl.loop(0, nloops, step=NBUF)
    def _(p):
        for b in range(NBUF):
            i = p + b
            srcv, dstv, typv, wrows, wv, gv = bufs[b]
            base = base0 + i * C

            @pl.when(i < NCH)
            def _():
                pltpu.make_async_copy(
                    winv_h.at[dstv], wrows, grsems[b]).wait()
                for k in range(C // 16):
                    sl = pl.ds(16 * k, 16)
                    tt = typv[sl]
                    wv[sl] = plsc.load_gather(wrows, [iot + 16 * k, tt])
                    gv[sl] = tt * N + srcv[sl]
                pltpu.async_copy(wv, w_h.at[pl.ds(base, C)], osems[b])
                pltpu.async_copy(gv, g_h.at[pl.ds(base, C)], osems[b])

            t3 = i + 3
            b3 = (b + 3) % NBUF

            @pl.when(t3 < NCH)
            def _():
                @pl.when(t3 >= NBUF)
                def _():
                    wait_out(b3, t3 - NBUF)

                issue_meta(b3, t3)

            t2 = i + 2
            b2 = (b + 2) % NBUF
            srcv2, dstv2, typv2, wrows2, wv2, gv2 = bufs[b2]

            @pl.when(t2 < NCH)
            def _():
                wait_meta(b2, t2)
                pltpu.async_copy(winv_h.at[dstv2], wrows2, grsems[b2])

    for b in range(NBUF):
        wait_out(b, 0)


def _sc_wg(winv, src, dst, typ):
    per_buf = [pltpu.VMEM((C,), jnp.int32),
               pltpu.VMEM((C,), jnp.int32),
               pltpu.VMEM((C,), jnp.int32),
               pltpu.VMEM((C, D), _F32),
               pltpu.VMEM((C,), _F32),
               pltpu.VMEM((C,), jnp.int32)]
    return pl.kernel(
        _sc_wg_body,
        out_type=(jax.ShapeDtypeStruct((E,), _F32),
                  jax.ShapeDtypeStruct((E,), jnp.int32)),
        mesh=_MESH,
        compiler_params=_SC_PARAMS,
        scratch_types=(
            per_buf * NBUF
            + [pltpu.SemaphoreType.DMA] * (3 * NBUF)
        ),
    )(winv, src, dst, typ)


def _sc_layer_body(y_h, g_h, w_h, dst_h, out_h, zb, acc, *rest):
    bufs = [rest[4 * b:4 * b + 4] for b in range(NBUF)]
    msems = rest[4 * NBUF:5 * NBUF]
    gsems = rest[5 * NBUF:6 * NBUF]
    ssems = rest[6 * NBUF:7 * NBUF]
    c, s, wid = _worker_ids()
    zeros16 = jnp.zeros((16,), _F32)

    @pl.loop(0, ZB)
    def _(i):
        for j in range(D // 16):
            zb[i, pl.ds(16 * j, 16)] = zeros16

    @pl.loop(0, RPT // ZB)
    def _(i):
        pltpu.sync_copy(zb, acc.at[pl.ds(s * RPT + i * ZB, ZB)])

    plsc.subcore_barrier()

    base0 = wid * EPT

    def issue_meta(b, i):
        gv, wv, dstv, yb = bufs[b]
        base = base0 + i * C
        pltpu.async_copy(g_h.at[pl.ds(base, C)], gv, msems[b])
        pltpu.async_copy(w_h.at[pl.ds(base, C)], wv, msems[b])
        pltpu.async_copy(dst_h.at[pl.ds(base, C)], dstv, msems[b])

    def wait_meta(b, i):
        gv, wv, dstv, yb = bufs[b]
        base = base0 + i * C
        pltpu.make_async_copy(g_h.at[pl.ds(base, C)], gv, msems[b]).wait()
        pltpu.make_async_copy(w_h.at[pl.ds(base, C)], wv, msems[b]).wait()
        pltpu.make_async_copy(dst_h.at[pl.ds(base, C)], dstv, msems[b]).wait()

    def scale(wv, yb):
        @pl.loop(0, C, unroll=8)
        def _(e):
            we = plsc.load_gather(wv, [jnp.full((16,), e, jnp.int32)])
            for j in range(D // 16):
                sl = pl.ds(16 * j, 16)
                yb[e, sl] = yb[e, sl] * we

    for t in range(3):
        issue_meta(t % NBUF, t)
    for t in range(2):
        b = t % NBUF
        gv, _, _, yb = bufs[b]
        wait_meta(b, t)
        pltpu.async_copy(y_h.at[gv], yb, gsems[b])

    nloops = (NCH + NBUF - 1) // NBUF * NBUF

    @pl.loop(0, nloops, step=NBUF)
    def _(p):
        for b in range(NBUF):
            i = p + b
            gv, wv, dstv, yb = bufs[b]

            @pl.when(i < NCH)
            def _():
                pltpu.make_async_copy(y_h.at[gv], yb, gsems[b]).wait()
                scale(wv, yb)
                pltpu.async_copy(yb, acc.at[dstv], ssems[b], add=True)

            t3 = i + 3
            b3 = (b + 3) % NBUF
            gv3, wv3, dstv3, yb3 = bufs[b3]

            @pl.when(t3 < NCH)
            def _():
                @pl.when(t3 >= NBUF)
                def _():
                    pltpu.make_async_copy(
                        yb3, acc.at[dstv3], ssems[b3]).wait()

                issue_meta(b3, t3)

            t2 = i + 2
            b2 = (b + 2) % NBUF
            gv2, wv2, dstv2, yb2 = bufs[b2]

            @pl.when(t2 < NCH)
            def _():
                wait_meta(b2, t2)
                pltpu.async_copy(y_h.at[gv2], yb2, gsems[b2])

    for b in range(NBUF):
        gv, wv, dstv, yb = bufs[b]
        pltpu.make_async_copy(yb, acc.at[dstv], ssems[b]).wait()

    plsc.subcore_barrier()
    pltpu.sync_copy(acc.at[pl.ds(s * RPT, RPT)],
                    out_h.at[c, pl.ds(s * RPT, RPT)])


def _sc_layer(yflat, g, w, dst):
    per_buf = [pltpu.VMEM((C,), jnp.int32),
               pltpu.VMEM((C,), _F32),
               pltpu.VMEM((C,), jnp.int32),
               pltpu.VMEM((C, D), _F32)]
    return pl.kernel(
        _sc_layer_body,
        out_type=jax.ShapeDtypeStruct((NC, NPAD, D), _F32),
        mesh=_MESH,
        compiler_params=_SC_PARAMS,
        scratch_types=(
            [pltpu.VMEM((ZB, D), _F32),
             pltpu.VMEM_SHARED((NPAD, D), _F32)]
            + per_buf * NBUF
            + [pltpu.SemaphoreType.DMA] * (3 * NBUF)
        ),
    )(yflat, g, w, dst)


def _tc_weights_body(comp_ref, basis_ref, out_ref):
    out_ref[0] = jnp.dot(comp_ref[0], basis_ref[0],
                         preferred_element_type=_F32)


def _tc_weights(comp_all, basis_all):
    return pl.pallas_call(
        _tc_weights_body,
        grid=(3,),
        in_specs=[
            pl.BlockSpec((1, R, NB), lambda i: (i, 0, 0)),
            pl.BlockSpec((1, NB, D * D), lambda i: (i, 0, 0)),
        ],
        out_specs=pl.BlockSpec((1, R, D * D), lambda i: (i, 0, 0)),
        out_shape=jax.ShapeDtypeStruct((3, R, D * D), _F32),
    )(comp_all, basis_all)


def _tc_mm_body(h_ref, w_ref, y_ref):
    h = h_ref[...]
    for r in range(R):
        y_ref[r] = jnp.dot(h, w_ref[r], preferred_element_type=_F32)


def _tc_mm(h, w3):
    return pl.pallas_call(
        _tc_mm_body,
        grid=(NBLK,),
        in_specs=[
            pl.BlockSpec((BN, D), lambda j: (j, 0)),
            pl.BlockSpec((R, D, D), lambda j: (0, 0, 0)),
        ],
        out_specs=pl.BlockSpec((R, BN, D), lambda j: (0, j, 0)),
        out_shape=jax.ShapeDtypeStruct((R, N, D), _F32),
    )(h, w3)


def _tc_winv_body(cnt_ref, out_ref):
    csum = cnt_ref[0] + cnt_ref[1]
    out_ref[...] = 1.0 / jnp.maximum(csum, 1.0)


def _tc_winv(cnts):
    return pl.pallas_call(
        _tc_winv_body,
        grid=(NS,),
        in_specs=[pl.BlockSpec((NC, RPT, D), lambda j: (0, j, 0))],
        out_specs=pl.BlockSpec((RPT, D), lambda j: (j, 0)),
        out_shape=jax.ShapeDtypeStruct((NPAD, D), _F32),
    )(cnts)


def _tc_dense_body(h_ref, root_ref, bias_ref, acc_ref, out_ref, *, relu):
    o = jnp.dot(h_ref[...], root_ref[...], preferred_element_type=_F32)
    o = o + bias_ref[...] + acc_ref[0] + acc_ref[1]
    if relu:
        o = jnp.maximum(o, 0.0)
    out_ref[...] = o


def _tc_dense(h, root, bias, acc, relu):
    return pl.pallas_call(
        functools.partial(_tc_dense_body, relu=relu),
        grid=(NBLK,),
        in_specs=[
            pl.BlockSpec((BN, D), lambda j: (j, 0)),
            pl.BlockSpec((D, D), lambda j: (0, 0)),
            pl.BlockSpec((1, D), lambda j: (0, 0)),
            pl.BlockSpec((NC, BN, D), lambda j: (0, j, 0)),
        ],
        out_specs=pl.BlockSpec((BN, D), lambda j: (j, 0)),
        out_shape=jax.ShapeDtypeStruct((N, D), _F32),
    )(h, root, bias, acc)


def _tc_dense_mm_body(h_ref, root_ref, bias_ref, acc_ref, w_ref,
                      hout_ref, y_ref):
    o = jnp.dot(h_ref[...], root_ref[...], preferred_element_type=_F32)
    o = o + bias_ref[...] + acc_ref[0] + acc_ref[1]
    o = jnp.maximum(o, 0.0)
    hout_ref[...] = o
    for r in range(R):
        y_ref[r] = jnp.dot(o, w_ref[r], preferred_element_type=_F32)


def _tc_dense_mm(h, root, bias, acc, w3):
    return pl.pallas_call(
        _tc_dense_mm_body,
        grid=(NBLK,),
        in_specs=[
            pl.BlockSpec((BN, D), lambda j: (j, 0)),
            pl.BlockSpec((D, D), lambda j: (0, 0)),
            pl.BlockSpec((1, D), lambda j: (0, 0)),
            pl.BlockSpec((NC, BN, D), lambda j: (0, j, 0)),
            pl.BlockSpec((R, D, D), lambda j: (0, 0, 0)),
        ],
        out_specs=[
            pl.BlockSpec((BN, D), lambda j: (j, 0)),
            pl.BlockSpec((R, BN, D), lambda j: (0, j, 0)),
        ],
        out_shape=[
            jax.ShapeDtypeStruct((N, D), _F32),
            jax.ShapeDtypeStruct((R, N, D), _F32),
        ],
    )(h, root, bias, acc, w3)


def kernel(x, edge_index, edge_type,
           comp1, basis1, root1, bias1,
           comp2, basis2, root2, bias2,
           comp3, basis3, root3, bias3):
    src = edge_index[0]
    dst = edge_index[1]

    comp_all = jnp.stack([comp1, comp2, comp3])
    basis_all = jnp.stack([basis1.reshape(NB, D * D),
                           basis2.reshape(NB, D * D),
                           basis3.reshape(NB, D * D)])
    w_all = _tc_weights(comp_all, basis_all).reshape(3, R, D, D)

    cnts = _sc_count(dst, edge_type)
    winv = _tc_winv(cnts)
    w_e, g_e = _sc_wg(winv, src, dst, edge_type)

    y = _tc_mm(x, w_all[0])
    acc = _sc_layer(y.reshape(R * N, D), g_e, w_e, dst)
    h1, y = _tc_dense_mm(x, root1, bias1.reshape(1, D), acc, w_all[1])
    acc = _sc_layer(y.reshape(R * N, D), g_e, w_e, dst)
    h2, y = _tc_dense_mm(h1, root2, bias2.reshape(1, D), acc, w_all[2])
    acc = _sc_layer(y.reshape(R * N, D), g_e, w_e, dst)
    return _tc_dense(h2, root3, bias3.reshape(1, D), acc, relu=False)
